# Initial kernel scaffold; baseline (speedup 1.0000x reference)
import functools

import jax
import jax.numpy as jnp
from jax import lax
from jax.experimental import pallas as pl
from jax.experimental.pallas import tpu as pltpu
from jax.experimental.pallas import tpu_sc as plsc

N, E, H, TD, RBF, L = 10000, 160000, 96, 32, 16, 4
NP = 10240
EP = 163840
DW = 112
PAD_ROW = NP - 2
NTILES, NCHUNK, CH = 32, 40, 128
EPT = NCHUNK * CH
BE = 1280
BN = 512
NPT = NP // 16

f32 = jnp.float32
i32 = jnp.int32
_dot = functools.partial(jnp.dot, preferred_element_type=jnp.float32)


_sc_mesh = plsc.VectorSubcoreMesh(
    core_axis_name="c", subcore_axis_name="s", num_cores=2, num_subcores=16)


@functools.partial(
    pl.kernel,
    out_type=(jax.ShapeDtypeStruct((EP, DW), f32),
              jax.ShapeDtypeStruct((EP, DW), f32)),
    mesh=_sc_mesh,
    scratch_types=[
        pltpu.VMEM((NCHUNK, CH), i32),
        pltpu.VMEM((NCHUNK, CH), i32),
        pltpu.VMEM((CH, DW), f32),
        pltpu.VMEM((CH, DW), f32),
        pltpu.SemaphoreType.DMA,
        pltpu.SemaphoreType.DMA,
    ],
)
def _sc_gather(tab_p, tab_q, src3, dst3, out_p, out_q,
               idx_s, idx_d, buf_p, buf_q, sem_p, sem_q):
    wid = lax.axis_index("c") * 16 + lax.axis_index("s")
    base = wid * EPT
    pltpu.sync_copy(src3.at[wid], idx_s)
    pltpu.sync_copy(dst3.at[wid], idx_d)

    def body(j, carry):
        dp = pltpu.async_copy(tab_p.at[idx_s.at[j]], buf_p, sem_p)
        dq = pltpu.async_copy(tab_q.at[idx_d.at[j]], buf_q, sem_q)
        dp.wait()
        pltpu.sync_copy(buf_p, out_p.at[pl.ds(base + j * CH, CH)])
        dq.wait()
        pltpu.sync_copy(buf_q, out_q.at[pl.ds(base + j * CH, CH)])
        return carry

    lax.fori_loop(0, NCHUNK, body, 0)


@functools.partial(
    pl.kernel,
    out_type=jax.ShapeDtypeStruct((2, NP, DW), f32),
    mesh=_sc_mesh,
    scratch_types=[
        pltpu.VMEM_SHARED((NP, DW), f32),
        pltpu.VMEM((NCHUNK, CH), i32),
        pltpu.VMEM((CH, DW), f32),
        pltpu.VMEM((CH, DW), f32),
    ],
)
def _sc_scatter(rows, src3, acc2, shared, idx_s, buf, zbuf):
    cid = lax.axis_index("c")
    sid = lax.axis_index("s")
    wid = cid * 16 + sid
    base = wid * EPT

    def zrow(r, carry):
        for c in range(DW // 16):
            zbuf[r, pl.ds(c * 16, 16)] = jnp.zeros((16,), f32)
        return carry

    lax.fori_loop(0, CH, zrow, 0)
    for c in range(NPT // CH):
        pltpu.sync_copy(zbuf, shared.at[pl.ds(sid * NPT + c * CH, CH)])
    plsc.subcore_barrier()

    pltpu.sync_copy(src3.at[wid], idx_s)

    def body(j, carry):
        pltpu.sync_copy(rows.at[pl.ds(base + j * CH, CH)], buf)
        pltpu.sync_copy(buf, shared.at[idx_s.at[j]], add=True)
        return carry

    lax.fori_loop(0, NCHUNK, body, 0)
    plsc.subcore_barrier()
    pltpu.sync_copy(shared.at[pl.ds(sid * NPT, NPT)],
                    acc2.at[cid, pl.ds(sid * NPT, NPT)])


def _time_body(sig, tw1t, tb1, tw2t, tb2, w1dt, b1c, out):
    s = sig[0:1, 0:1]
    i = lax.broadcasted_iota(f32, (8, TD // 2), 1)
    freqs = jnp.exp(i * (jnp.log(1000.0) / (TD // 2 - 1)))
    args = s * freqs
    emb = jnp.concatenate([jnp.sin(args), jnp.cos(args)], axis=1)
    te = jax.nn.silu(_dot(emb, tw1t[...]) + tb1[...])
    te = _dot(te, tw2t[...]) + tb2[...]
    out[...] = _dot(te, w1dt[...]) + b1c[...]


def _pro_body(ti, tg, nt, sp, x8,
              pet, net, let, modt, pw1r, pb1, pw2t, pb2, a0t, b0t,
              h_o, p_o, q_o):
    tiv = ti[...]
    oh25 = (tiv == lax.broadcasted_iota(i32, (BN, 32), 1)).astype(f32)
    pe = _dot(oh25, pet[...])
    ohn = (jnp.maximum(tiv, 0)
           == lax.broadcasted_iota(i32, (BN, 8), 1)).astype(f32)
    ne = _dot(ohn, net[...])
    oh64 = (tiv == lax.broadcasted_iota(i32, (BN, 64), 1)).astype(f32)
    le = _dot(oh64, let[...])
    tgv = tg[...]
    feats = jnp.where(tgv == 0, pe, 0.0)
    feats = jnp.where(tgv == 1, ne, feats)
    feats = jnp.where(tgv == 2, le, feats)
    ohm = (nt[...] == lax.broadcasted_iota(i32, (BN, 8), 1)).astype(f32)
    feats = feats + _dot(ohm, modt[...])
    p = jax.nn.silu(sp[...] * pw1r[0:1, :] + pb1[0:1, :])
    feats = feats + _dot(p, pw2t[...]) + pb2[0:1, :]
    h_o[...] = feats
    g = _dot(feats, a0t[...])
    k = _dot(feats, b0t[...])
    xs = x8[...][:, 0:3]
    z13 = jnp.zeros((BN, 13), f32)
    p_o[...] = jnp.concatenate([g, xs, z13], axis=1)
    q_o[...] = jnp.concatenate([k, xs, z13], axis=1)


def _edge_body(pr, qr, et, w2t, cw1t, drbf, ctab, misc, out):
    p = pr[...]
    q = qr[...]
    g = p[:, :H] + q[:, :H]
    rel = p[:, H:H + 3] - q[:, H:H + 3]
    d2 = jnp.sum(rel * rel, axis=1, keepdims=True)
    dist = jnp.maximum(jnp.sqrt(d2), 1e-6)
    centers = lax.broadcasted_iota(f32, (BE, RBF), 1) * (12.0 / (RBF - 1))
    diff = dist - centers
    rbf = jnp.exp(-0.5 * diff * diff)
    ohe = (et[...] == lax.broadcasted_iota(i32, (BE, 8), 1)).astype(f32)
    pre = g + _dot(rbf, drbf[...]) + _dot(ohe, ctab[...]) + misc[0:1, :]
    m1 = jax.nn.silu(pre)
    msgs = jax.nn.silu(_dot(m1, w2t[...]) + misc[1:2, :])
    c1 = jax.nn.silu(_dot(msgs, cw1t[...]) + misc[2:3, :])
    cs = jnp.sum(c1 * misc[3:4, :], axis=1, keepdims=True) + misc[4:5, 0:1]
    w = cs / (dist + 1.0)
    out[...] = jnp.concatenate(
        [msgs, w * rel, jnp.zeros((BE, 13), f32)], axis=1)


def _node_common(h, x8, acc, n1ht, n1ft, n2t, nmisc):
    a = acc[0] + acc[1]
    fa = a[:, :H]
    ca = a[:, H:H + 3]
    hv = h[...]
    t1 = jax.nn.silu(_dot(hv, n1ht[...]) + _dot(fa, n1ft[...])
                     + nmisc[0:1, :])
    hn = hv + _dot(t1, n2t[...]) + nmisc[1:2, :]
    xn = x8[...] + jnp.concatenate([ca, jnp.zeros((BN, 5), f32)], axis=1)
    return hn, xn


def _node_body(h, x8, acc, n1ht, n1ft, n2t, nmisc, a1t, b1t,
               h_o, x_o, p_o, q_o):
    hn, xn = _node_common(h, x8, acc, n1ht, n1ft, n2t, nmisc)
    h_o[...] = hn
    x_o[...] = xn
    g = _dot(hn, a1t[...])
    k = _dot(hn, b1t[...])
    xs = xn[:, 0:3]
    z13 = jnp.zeros((BN, 13), f32)
    p_o[...] = jnp.concatenate([g, xs, z13], axis=1)
    q_o[...] = jnp.concatenate([k, xs, z13], axis=1)


def _node_final_body(h, x8, acc, n1ht, n1ft, n2t, nmisc, ow1t, ow2t8, ob8,
                     res_o):
    hn, xn = _node_common(h, x8, acc, n1ht, n1ft, n2t, nmisc)
    d1 = jax.nn.silu(_dot(hn, ow1t[...]) + nmisc[2:3, :])
    res_o[...] = xn + _dot(d1, ow2t8[...]) + ob8[0:1, :]


def _full(shape):
    return pl.BlockSpec(shape, lambda i: (0,) * len(shape))


def _rows(shape):
    return pl.BlockSpec(shape, lambda i: (i,) + (0,) * (len(shape) - 1))


_time_call = pl.pallas_call(
    _time_body,
    out_shape=jax.ShapeDtypeStruct((8, L * H), f32),
)

_pro_call = pl.pallas_call(
    _pro_body,
    grid=(NP // BN,),
    in_specs=[
        _rows((BN, 1)), _rows((BN, 1)), _rows((BN, 1)), _rows((BN, 1)),
        _rows((BN, 8)),
        _full((32, H)), _full((8, H)), _full((64, H)), _full((8, H)),
        _full((8, H)), _full((8, H)), _full((H, H)), _full((8, H)),
        _full((H, H)), _full((H, H)),
    ],
    out_specs=[_rows((BN, H)), _rows((BN, DW)), _rows((BN, DW))],
    out_shape=[jax.ShapeDtypeStruct((NP, H), f32),
               jax.ShapeDtypeStruct((NP, DW), f32),
               jax.ShapeDtypeStruct((NP, DW), f32)],
)

_edge_call = pl.pallas_call(
    _edge_body,
    grid=(EP // BE,),
    in_specs=[
        _rows((BE, DW)), _rows((BE, DW)), _rows((BE, 1)),
        _full((H, H)), _full((H, H)), _full((RBF, H)), _full((8, H)),
        _full((8, H)),
    ],
    out_specs=_rows((BE, DW)),
    out_shape=jax.ShapeDtypeStruct((EP, DW), f32),
)

_acc_spec = pl.BlockSpec((2, BN, DW), lambda i: (0, i, 0))

_node_call = pl.pallas_call(
    _node_body,
    grid=(NP // BN,),
    in_specs=[
        _rows((BN, H)), _rows((BN, 8)), _acc_spec,
        _full((H, H)), _full((H, H)), _full((H, H)), _full((8, H)),
        _full((H, H)), _full((H, H)),
    ],
    out_specs=[_rows((BN, H)), _rows((BN, 8)),
               _rows((BN, DW)), _rows((BN, DW))],
    out_shape=[jax.ShapeDtypeStruct((NP, H), f32),
               jax.ShapeDtypeStruct((NP, 8), f32),
               jax.ShapeDtypeStruct((NP, DW), f32),
               jax.ShapeDtypeStruct((NP, DW), f32)],
)

_node_final_call = pl.pallas_call(
    _node_final_body,
    grid=(NP // BN,),
    in_specs=[
        _rows((BN, H)), _rows((BN, 8)), _acc_spec,
        _full((H, H)), _full((H, H)), _full((H, H)), _full((8, H)),
        _full((H, H)), _full((H, 8)), _full((8, 8)),
    ],
    out_specs=_rows((BN, 8)),
    out_shape=jax.ShapeDtypeStruct((NP, 8), f32),
)


def kernel(noisy_coords, sigma, token_groups, token_indices, node_types,
           sequence_positions, edge_index, edge_types, params):
    layers = params['layers']

    def bc8(v):
        return jnp.broadcast_to(v[None, :], (8, v.shape[0]))

    x8 = jnp.pad(noisy_coords[0].astype(f32), ((0, NP - N), (0, 5)))
    ti2 = jnp.pad(token_indices.astype(i32), (0, NP - N))[:, None]
    tg2 = jnp.pad(token_groups.astype(i32), (0, NP - N))[:, None]
    nt2 = jnp.pad(node_types.astype(i32), (0, NP - N))[:, None]
    sp2 = jnp.pad(sequence_positions.astype(f32), (0, NP - N))[:, None]
    src = edge_index[:, 0].astype(i32)
    dst = edge_index[:, 1].astype(i32)
    idx_pad = jnp.full((EP - E,), PAD_ROW, i32)
    src3 = jnp.concatenate([src, idx_pad]).reshape(NTILES, NCHUNK, CH)
    dst3 = jnp.concatenate([dst, idx_pad]).reshape(NTILES, NCHUNK, CH)
    et2 = jnp.pad(edge_types.astype(i32), (0, EP - E))[:, None]

    w1dt = jnp.concatenate(
        [lp['edge_W1'][:, 288:320].T for lp in layers], axis=1)
    b1c = jnp.broadcast_to(
        jnp.concatenate([lp['edge_b1'] for lp in layers])[None, :],
        (8, L * H))
    sig8 = jnp.broadcast_to(sigma.reshape(1, 1).astype(f32), (8, 128))
    tv = _time_call(sig8, params['time_W1'].T, bc8(params['time_b1']),
                    params['time_W2'].T, bc8(params['time_b2']), w1dt, b1c)

    at = [lp['edge_W1'][:, 0:H].T for lp in layers]
    bt = [lp['edge_W1'][:, H:2 * H].T for lp in layers]
    drbf = [lp['edge_W1'][:, 3 * H + TD:].T for lp in layers]
    ctab = [jnp.pad(_dot(lp['edge_emb'], lp['edge_W1'][:, 2 * H:3 * H].T),
                    ((0, 3), (0, 0))) for lp in layers]
    w2t = [lp['edge_W2'].T for lp in layers]
    cw1t = [lp['coord_W1'].T for lp in layers]
    miscs = []
    for l, lp in enumerate(layers):
        miscs.append(jnp.stack([
            tv[0, l * H:(l + 1) * H],
            lp['edge_b2'],
            lp['coord_b1'],
            lp['coord_W2'][0],
            jnp.full((H,), lp['coord_b2'][0]),
            jnp.zeros((H,), f32), jnp.zeros((H,), f32), jnp.zeros((H,), f32),
        ]))
    n1ht = [lp['node_W1'][:, 0:H].T for lp in layers]
    n1ft = [lp['node_W1'][:, H:].T for lp in layers]
    n2t = [lp['node_W2'].T for lp in layers]
    nmiscs = []
    for lp in layers:
        nmiscs.append(jnp.stack([
            lp['node_b1'], lp['node_b2'], params['out_b1'],
            jnp.zeros((H,), f32), jnp.zeros((H,), f32),
            jnp.zeros((H,), f32), jnp.zeros((H,), f32), jnp.zeros((H,), f32),
        ]))
    ow1t = params['out_W1'].T
    ow2t8 = jnp.pad(params['out_W2'].T, ((0, 0), (0, 5)))
    ob8 = jnp.broadcast_to(jnp.pad(params['out_b2'], (0, 5))[None, :], (8, 8))

    h, tab_p, tab_q = _pro_call(
        ti2, tg2, nt2, sp2, x8,
        jnp.pad(params['protein_emb'], ((0, 7), (0, 0))),
        params['nucleotide_emb'],
        params['ligand_emb'],
        jnp.pad(params['modality_emb'], ((0, 5), (0, 0))),
        bc8(params['pos_W1'][:, 0]), bc8(params['pos_b1']),
        params['pos_W2'].T, bc8(params['pos_b2']),
        at[0], bt[0])

    for l in range(L):
        rows_p, rows_q = _sc_gather(tab_p, tab_q, src3, dst3)
        rows = _edge_call(rows_p, rows_q, et2, w2t[l], cw1t[l], drbf[l],
                          ctab[l], miscs[l])
        acc2 = _sc_scatter(rows, src3)
        if l < L - 1:
            h, x8, tab_p, tab_q = _node_call(
                h, x8, acc2, n1ht[l], n1ft[l], n2t[l], nmiscs[l],
                at[l + 1], bt[l + 1])
        else:
            res = _node_final_call(
                h, x8, acc2, n1ht[l], n1ft[l], n2t[l], nmiscs[l],
                ow1t, ow2t8, ob8)

    return res[:N, :3][None]


# speedup vs baseline: 2.7692x; 2.7692x over previous
import functools

import jax
import jax.numpy as jnp
from jax import lax
from jax.experimental import pallas as pl
from jax.experimental.pallas import tpu as pltpu
from jax.experimental.pallas import tpu_sc as plsc

N, E, H, TD, RBF, L = 10000, 160000, 96, 32, 16, 4
NP = 10240
EP = 163840
DW = 128
PADW = DW - H - 3
PAD_ROW = NP - 2
NTILES, NCHUNK, CH = 32, 40, 128
EPT = NCHUNK * CH
BE = 1280
BN = 512
NPT = NP // 16

f32 = jnp.float32
i32 = jnp.int32
_dot = functools.partial(jnp.dot, preferred_element_type=jnp.float32)


@functools.lru_cache(maxsize=None)
def _sc_mesh():
    return plsc.VectorSubcoreMesh(
        core_axis_name="c", subcore_axis_name="s",
        num_cores=2, num_subcores=16)


@functools.lru_cache(maxsize=None)
def _sc_gather_call():
    @functools.partial(
        pl.kernel,
        out_type=(jax.ShapeDtypeStruct((EP, DW), f32),
                  jax.ShapeDtypeStruct((EP, DW), f32)),
        mesh=_sc_mesh(),
        scratch_types=[
            pltpu.VMEM((NCHUNK, CH), i32),
            pltpu.VMEM((NCHUNK, CH), i32),
            pltpu.VMEM((CH, DW), f32),
            pltpu.VMEM((CH, DW), f32),
            pltpu.SemaphoreType.DMA,
            pltpu.SemaphoreType.DMA,
        ],
    )
    def _sc_gather(tab_p, tab_q, src3, dst3, out_p, out_q,
                   idx_s, idx_d, buf_p, buf_q, sem_p, sem_q):
        wid = lax.axis_index("c") * 16 + lax.axis_index("s")
        base = wid * EPT
        pltpu.sync_copy(src3.at[wid], idx_s)
        pltpu.sync_copy(dst3.at[wid], idx_d)

        def body(j, carry):
            dp = pltpu.async_copy(tab_p.at[idx_s.at[j]], buf_p, sem_p)
            dq = pltpu.async_copy(tab_q.at[idx_d.at[j]], buf_q, sem_q)
            dp.wait()
            pltpu.sync_copy(buf_p, out_p.at[pl.ds(base + j * CH, CH)])
            dq.wait()
            pltpu.sync_copy(buf_q, out_q.at[pl.ds(base + j * CH, CH)])
            return carry

        lax.fori_loop(0, NCHUNK, body, 0)

    return _sc_gather


def _gather(tab_p, tab_q, src3, dst3):
    return _sc_gather_call()(tab_p, tab_q, src3, dst3)


@functools.lru_cache(maxsize=None)
def _sc_scatter_call():
    @functools.partial(
        pl.kernel,
        out_type=jax.ShapeDtypeStruct((2, NP, DW), f32),
        mesh=_sc_mesh(),
        scratch_types=[
            pltpu.VMEM_SHARED((NP, DW), f32),
            pltpu.VMEM((NCHUNK, CH), i32),
            pltpu.VMEM((CH, DW), f32),
            pltpu.VMEM((CH, DW), f32),
        ],
    )
    def _sc_scatter(rows, src3, acc2, shared, idx_s, buf, zbuf):
        cid = lax.axis_index("c")
        sid = lax.axis_index("s")
        wid = cid * 16 + sid
        base = wid * EPT

        def zrow(r, carry):
            for c in range(DW // 16):
                zbuf[r, pl.ds(c * 16, 16)] = jnp.zeros((16,), f32)
            return carry

        lax.fori_loop(0, CH, zrow, 0)
        for c in range(NPT // CH):
            pltpu.sync_copy(zbuf, shared.at[pl.ds(sid * NPT + c * CH, CH)])
        plsc.subcore_barrier()

        pltpu.sync_copy(src3.at[wid], idx_s)

        def body(j, carry):
            pltpu.sync_copy(rows.at[pl.ds(base + j * CH, CH)], buf)
            pltpu.sync_copy(buf, shared.at[idx_s.at[j]], add=True)
            return carry

        lax.fori_loop(0, NCHUNK, body, 0)
        plsc.subcore_barrier()
        pltpu.sync_copy(shared.at[pl.ds(sid * NPT, NPT)],
                        acc2.at[cid, pl.ds(sid * NPT, NPT)])

    return _sc_scatter


def _scatter(rows, src3):
    return _sc_scatter_call()(rows, src3)


def _time_body(sig, tw1t, tb1, tw2t, tb2, w1dt, b1c, out):
    s = sig[0:1, 0:1]
    i = lax.broadcasted_iota(i32, (8, TD // 2), 1).astype(f32)
    freqs = jnp.exp(i * (jnp.log(1000.0) / (TD // 2 - 1)))
    args = s * freqs
    emb = jnp.concatenate([jnp.sin(args), jnp.cos(args)], axis=1)
    te = jax.nn.silu(_dot(emb, tw1t[...]) + tb1[...])
    te = _dot(te, tw2t[...]) + tb2[...]
    out[...] = _dot(te, w1dt[...]) + b1c[...]


def _pro_body(ti, tg, nt, sp, x8,
              pet, net, let, modt, pw1r, pb1, pw2t, pb2, a0t, b0t,
              h_o, p_o, q_o):
    tiv = ti[...]
    oh25 = (tiv == lax.broadcasted_iota(i32, (BN, 32), 1)).astype(f32)
    pe = _dot(oh25, pet[...])
    ohn = (jnp.maximum(tiv, 0)
           == lax.broadcasted_iota(i32, (BN, 8), 1)).astype(f32)
    ne = _dot(ohn, net[...])
    oh64 = (tiv == lax.broadcasted_iota(i32, (BN, 64), 1)).astype(f32)
    le = _dot(oh64, let[...])
    tgv = tg[...]
    feats = jnp.where(tgv == 0, pe, 0.0)
    feats = jnp.where(tgv == 1, ne, feats)
    feats = jnp.where(tgv == 2, le, feats)
    ohm = (nt[...] == lax.broadcasted_iota(i32, (BN, 8), 1)).astype(f32)
    feats = feats + _dot(ohm, modt[...])
    p = jax.nn.silu(sp[...] * pw1r[0:1, :] + pb1[0:1, :])
    feats = feats + _dot(p, pw2t[...]) + pb2[0:1, :]
    h_o[...] = feats
    g = _dot(feats, a0t[...])
    k = _dot(feats, b0t[...])
    xs = x8[...][:, 0:3]
    z13 = jnp.zeros((BN, PADW), f32)
    p_o[...] = jnp.concatenate([g, xs, z13], axis=1)
    q_o[...] = jnp.concatenate([k, xs, z13], axis=1)


def _edge_body(pr, qr, et, w2t, cw1t, drbf, ctab, misc, out):
    p = pr[...]
    q = qr[...]
    g = p[:, :H] + q[:, :H]
    rel = p[:, H:H + 3] - q[:, H:H + 3]
    d2 = jnp.sum(rel * rel, axis=1, keepdims=True)
    dist = jnp.maximum(jnp.sqrt(d2), 1e-6)
    centers = (lax.broadcasted_iota(i32, (BE, RBF), 1).astype(f32)
               * (12.0 / (RBF - 1)))
    diff = dist - centers
    rbf = jnp.exp(-0.5 * diff * diff)
    ohe = (et[...] == lax.broadcasted_iota(i32, (BE, 8), 1)).astype(f32)
    pre = g + _dot(rbf, drbf[...]) + _dot(ohe, ctab[...]) + misc[0:1, :]
    m1 = jax.nn.silu(pre)
    msgs = jax.nn.silu(_dot(m1, w2t[...]) + misc[1:2, :])
    c1 = jax.nn.silu(_dot(msgs, cw1t[...]) + misc[2:3, :])
    cs = jnp.sum(c1 * misc[3:4, :], axis=1, keepdims=True) + misc[4:5, 0:1]
    w = cs / (dist + 1.0)
    out[...] = jnp.concatenate(
        [msgs, w * rel, jnp.zeros((BE, PADW), f32)], axis=1)


def _node_common(h, x8, acc, n1ht, n1ft, n2t, nmisc):
    a = acc[0] + acc[1]
    fa = a[:, :H]
    ca = a[:, H:H + 3]
    hv = h[...]
    t1 = jax.nn.silu(_dot(hv, n1ht[...]) + _dot(fa, n1ft[...])
                     + nmisc[0:1, :])
    hn = hv + _dot(t1, n2t[...]) + nmisc[1:2, :]
    xn = x8[...] + jnp.concatenate([ca, jnp.zeros((BN, 5), f32)], axis=1)
    return hn, xn


def _node_body(h, x8, acc, n1ht, n1ft, n2t, nmisc, a1t, b1t,
               h_o, x_o, p_o, q_o):
    hn, xn = _node_common(h, x8, acc, n1ht, n1ft, n2t, nmisc)
    h_o[...] = hn
    x_o[...] = xn
    g = _dot(hn, a1t[...])
    k = _dot(hn, b1t[...])
    xs = xn[:, 0:3]
    z13 = jnp.zeros((BN, PADW), f32)
    p_o[...] = jnp.concatenate([g, xs, z13], axis=1)
    q_o[...] = jnp.concatenate([k, xs, z13], axis=1)


def _node_final_body(h, x8, acc, n1ht, n1ft, n2t, nmisc, ow1t, ow2t8, ob8,
                     res_o):
    hn, xn = _node_common(h, x8, acc, n1ht, n1ft, n2t, nmisc)
    d1 = jax.nn.silu(_dot(hn, ow1t[...]) + nmisc[2:3, :])
    res_o[...] = xn + _dot(d1, ow2t8[...]) + ob8[0:1, :]


def _full(shape):
    return pl.BlockSpec(shape, lambda i: (0,) * len(shape))


def _rows(shape):
    return pl.BlockSpec(shape, lambda i: (i,) + (0,) * (len(shape) - 1))


_time_call = pl.pallas_call(
    _time_body,
    out_shape=jax.ShapeDtypeStruct((8, L * H), f32),
)

_pro_call = pl.pallas_call(
    _pro_body,
    grid=(NP // BN,),
    in_specs=[
        _rows((BN, 1)), _rows((BN, 1)), _rows((BN, 1)), _rows((BN, 1)),
        _rows((BN, 8)),
        _full((32, H)), _full((8, H)), _full((64, H)), _full((8, H)),
        _full((8, H)), _full((8, H)), _full((H, H)), _full((8, H)),
        _full((H, H)), _full((H, H)),
    ],
    out_specs=[_rows((BN, H)), _rows((BN, DW)), _rows((BN, DW))],
    out_shape=[jax.ShapeDtypeStruct((NP, H), f32),
               jax.ShapeDtypeStruct((NP, DW), f32),
               jax.ShapeDtypeStruct((NP, DW), f32)],
)

_edge_call = pl.pallas_call(
    _edge_body,
    grid=(EP // BE,),
    in_specs=[
        _rows((BE, DW)), _rows((BE, DW)), _rows((BE, 1)),
        _full((H, H)), _full((H, H)), _full((RBF, H)), _full((8, H)),
        _full((8, H)),
    ],
    out_specs=_rows((BE, DW)),
    out_shape=jax.ShapeDtypeStruct((EP, DW), f32),
)

_acc_spec = pl.BlockSpec((2, BN, DW), lambda i: (0, i, 0))

_node_call = pl.pallas_call(
    _node_body,
    grid=(NP // BN,),
    in_specs=[
        _rows((BN, H)), _rows((BN, 8)), _acc_spec,
        _full((H, H)), _full((H, H)), _full((H, H)), _full((8, H)),
        _full((H, H)), _full((H, H)),
    ],
    out_specs=[_rows((BN, H)), _rows((BN, 8)),
               _rows((BN, DW)), _rows((BN, DW))],
    out_shape=[jax.ShapeDtypeStruct((NP, H), f32),
               jax.ShapeDtypeStruct((NP, 8), f32),
               jax.ShapeDtypeStruct((NP, DW), f32),
               jax.ShapeDtypeStruct((NP, DW), f32)],
)

_node_final_call = pl.pallas_call(
    _node_final_body,
    grid=(NP // BN,),
    in_specs=[
        _rows((BN, H)), _rows((BN, 8)), _acc_spec,
        _full((H, H)), _full((H, H)), _full((H, H)), _full((8, H)),
        _full((H, H)), _full((H, 8)), _full((8, 8)),
    ],
    out_specs=_rows((BN, 8)),
    out_shape=jax.ShapeDtypeStruct((NP, 8), f32),
)


def kernel(noisy_coords, sigma, token_groups, token_indices, node_types,
           sequence_positions, edge_index, edge_types, params):
    layers = params['layers']

    def bc8(v):
        return jnp.broadcast_to(v[None, :], (8, v.shape[0]))

    x8 = jnp.pad(noisy_coords[0].astype(f32), ((0, NP - N), (0, 5)))
    ti2 = jnp.pad(token_indices.astype(i32), (0, NP - N))[:, None]
    tg2 = jnp.pad(token_groups.astype(i32), (0, NP - N))[:, None]
    nt2 = jnp.pad(node_types.astype(i32), (0, NP - N))[:, None]
    sp2 = jnp.pad(sequence_positions.astype(f32), (0, NP - N))[:, None]
    src = edge_index[:, 0].astype(i32)
    dst = edge_index[:, 1].astype(i32)
    idx_pad = jnp.full((EP - E,), PAD_ROW, i32)
    src3 = jnp.concatenate([src, idx_pad]).reshape(NTILES, NCHUNK, CH)
    dst3 = jnp.concatenate([dst, idx_pad]).reshape(NTILES, NCHUNK, CH)
    et2 = jnp.pad(edge_types.astype(i32), (0, EP - E))[:, None]

    w1dt = jnp.concatenate(
        [lp['edge_W1'][:, 288:320].T for lp in layers], axis=1)
    b1c = jnp.broadcast_to(
        jnp.concatenate([lp['edge_b1'] for lp in layers])[None, :],
        (8, L * H))
    sig8 = jnp.broadcast_to(sigma.reshape(1, 1).astype(f32), (8, 128))
    tv = _time_call(sig8, params['time_W1'].T, bc8(params['time_b1']),
                    params['time_W2'].T, bc8(params['time_b2']), w1dt, b1c)

    at = [lp['edge_W1'][:, 0:H].T for lp in layers]
    bt = [lp['edge_W1'][:, H:2 * H].T for lp in layers]
    drbf = [lp['edge_W1'][:, 3 * H + TD:].T for lp in layers]
    ctab = [jnp.pad(_dot(lp['edge_emb'], lp['edge_W1'][:, 2 * H:3 * H].T),
                    ((0, 3), (0, 0))) for lp in layers]
    w2t = [lp['edge_W2'].T for lp in layers]
    cw1t = [lp['coord_W1'].T for lp in layers]
    miscs = []
    for l, lp in enumerate(layers):
        miscs.append(jnp.stack([
            tv[0, l * H:(l + 1) * H],
            lp['edge_b2'],
            lp['coord_b1'],
            lp['coord_W2'][0],
            jnp.full((H,), lp['coord_b2'][0]),
            jnp.zeros((H,), f32), jnp.zeros((H,), f32), jnp.zeros((H,), f32),
        ]))
    n1ht = [lp['node_W1'][:, 0:H].T for lp in layers]
    n1ft = [lp['node_W1'][:, H:].T for lp in layers]
    n2t = [lp['node_W2'].T for lp in layers]
    nmiscs = []
    for lp in layers:
        nmiscs.append(jnp.stack([
            lp['node_b1'], lp['node_b2'], params['out_b1'],
            jnp.zeros((H,), f32), jnp.zeros((H,), f32),
            jnp.zeros((H,), f32), jnp.zeros((H,), f32), jnp.zeros((H,), f32),
        ]))
    ow1t = params['out_W1'].T
    ow2t8 = jnp.pad(params['out_W2'].T, ((0, 0), (0, 5)))
    ob8 = jnp.broadcast_to(jnp.pad(params['out_b2'], (0, 5))[None, :], (8, 8))

    h, tab_p, tab_q = _pro_call(
        ti2, tg2, nt2, sp2, x8,
        jnp.pad(params['protein_emb'], ((0, 7), (0, 0))),
        params['nucleotide_emb'],
        params['ligand_emb'],
        jnp.pad(params['modality_emb'], ((0, 5), (0, 0))),
        bc8(params['pos_W1'][:, 0]), bc8(params['pos_b1']),
        params['pos_W2'].T, bc8(params['pos_b2']),
        at[0], bt[0])

    for l in range(L):
        rows_p, rows_q = _gather(tab_p, tab_q, src3, dst3)
        rows = _edge_call(rows_p, rows_q, et2, w2t[l], cw1t[l], drbf[l],
                          ctab[l], miscs[l])
        acc2 = _scatter(rows, src3)
        if l < L - 1:
            h, x8, tab_p, tab_q = _node_call(
                h, x8, acc2, n1ht[l], n1ft[l], n2t[l], nmiscs[l],
                at[l + 1], bt[l + 1])
        else:
            res = _node_final_call(
                h, x8, acc2, n1ht[l], n1ft[l], n2t[l], nmiscs[l],
                ow1t, ow2t8, ob8)

    return res[:N, :3][None]


# speedup vs baseline: 3.0890x; 1.1155x over previous
import functools

import jax
import jax.numpy as jnp
from jax import lax
from jax.experimental import pallas as pl
from jax.experimental.pallas import tpu as pltpu
from jax.experimental.pallas import tpu_sc as plsc

N, E, H, TD, RBF, L = 10000, 160000, 96, 32, 16, 4
NP = 10240
EP = 163840
DW = 128
PADW = DW - H - 3
PAD_ROW = NP - 2
NTILES, NCHUNK, CH = 32, 40, 128
EPT = NCHUNK * CH
BE = 1280
BN = 512
NPT = NP // 16

f32 = jnp.float32
i32 = jnp.int32
_dot = functools.partial(jnp.dot, preferred_element_type=jnp.float32)


@functools.lru_cache(maxsize=None)
def _sc_mesh():
    return plsc.VectorSubcoreMesh(
        core_axis_name="c", subcore_axis_name="s",
        num_cores=2, num_subcores=16)


@functools.lru_cache(maxsize=None)
def _sc_gather_call():
    @functools.partial(
        pl.kernel,
        out_type=(jax.ShapeDtypeStruct((EP, DW), f32),
                  jax.ShapeDtypeStruct((EP, DW), f32)),
        mesh=_sc_mesh(),
        scratch_types=[
            pltpu.VMEM((NCHUNK, CH), i32),
            pltpu.VMEM((NCHUNK, CH), i32),
            pltpu.VMEM((CH, DW), f32),
            pltpu.VMEM((CH, DW), f32),
            pltpu.VMEM((CH, DW), f32),
            pltpu.VMEM((CH, DW), f32),
        ] + [pltpu.SemaphoreType.DMA] * 8,
    )
    def _sc_gather(tab_p, tab_q, src3, dst3, out_p, out_q,
                   idx_s, idx_d, bp0, bq0, bp1, bq1,
                   sgp0, sgq0, sgp1, sgq1, ssp0, ssq0, ssp1, ssq1):
        wid = lax.axis_index("c") * 16 + lax.axis_index("s")
        base = wid * EPT
        pltpu.sync_copy(src3.at[wid], idx_s)
        pltpu.sync_copy(dst3.at[wid], idx_d)

        def wait64(sem):
            pltpu.make_async_copy(out_p.at[pl.ds(0, CH)], bp0, sem).wait()

        pltpu.async_copy(tab_p.at[idx_s.at[0]], bp0, sgp0)
        pltpu.async_copy(tab_q.at[idx_d.at[0]], bq0, sgq0)

        def body(i, carry):
            j0 = 2 * i

            @pl.when(i > 0)
            def _():
                wait64(ssp1)
                wait64(ssq1)

            pltpu.async_copy(tab_p.at[idx_s.at[j0 + 1]], bp1, sgp1)
            pltpu.async_copy(tab_q.at[idx_d.at[j0 + 1]], bq1, sgq1)
            wait64(sgp0)
            wait64(sgq0)
            pltpu.async_copy(bp0, out_p.at[pl.ds(base + j0 * CH, CH)], ssp0)
            pltpu.async_copy(bq0, out_q.at[pl.ds(base + j0 * CH, CH)], ssq0)

            @pl.when(i < NCHUNK // 2 - 1)
            def _():
                wait64(ssp0)
                wait64(ssq0)
                pltpu.async_copy(tab_p.at[idx_s.at[j0 + 2]], bp0, sgp0)
                pltpu.async_copy(tab_q.at[idx_d.at[j0 + 2]], bq0, sgq0)

            wait64(sgp1)
            wait64(sgq1)
            pltpu.async_copy(bp1, out_p.at[pl.ds(base + (j0 + 1) * CH, CH)],
                             ssp1)
            pltpu.async_copy(bq1, out_q.at[pl.ds(base + (j0 + 1) * CH, CH)],
                             ssq1)
            return carry

        lax.fori_loop(0, NCHUNK // 2, body, 0)
        wait64(ssp0)
        wait64(ssq0)
        wait64(ssp1)
        wait64(ssq1)

    return _sc_gather


def _gather(tab_p, tab_q, src3, dst3):
    return _sc_gather_call()(tab_p, tab_q, src3, dst3)


@functools.lru_cache(maxsize=None)
def _sc_scatter_call():
    @functools.partial(
        pl.kernel,
        out_type=jax.ShapeDtypeStruct((2, NP, DW), f32),
        mesh=_sc_mesh(),
        scratch_types=[
            pltpu.VMEM_SHARED((NP, DW), f32),
            pltpu.VMEM((NCHUNK, CH), i32),
            pltpu.VMEM((CH, DW), f32),
            pltpu.VMEM((CH, DW), f32),
            pltpu.SemaphoreType.DMA,
            pltpu.SemaphoreType.DMA,
        ],
    )
    def _sc_scatter(rows, src3, acc2, shared, idx_s, b0, b1, sl0, sl1):
        cid = lax.axis_index("c")
        sid = lax.axis_index("s")
        wid = cid * 16 + sid
        base = wid * EPT

        def zrow(r, carry):
            for c in range(DW // 16):
                b0[r, pl.ds(c * 16, 16)] = jnp.zeros((16,), f32)
            return carry

        lax.fori_loop(0, CH, zrow, 0)
        for c in range(NPT // CH):
            pltpu.sync_copy(b0, shared.at[pl.ds(sid * NPT + c * CH, CH)])
        plsc.subcore_barrier()

        pltpu.sync_copy(src3.at[wid], idx_s)

        def wait64(sem):
            pltpu.make_async_copy(rows.at[pl.ds(0, CH)], b0, sem).wait()

        pltpu.async_copy(rows.at[pl.ds(base, CH)], b0, sl0)

        def body(i, carry):
            j0 = 2 * i
            pltpu.async_copy(rows.at[pl.ds(base + (j0 + 1) * CH, CH)], b1,
                             sl1)
            wait64(sl0)
            pltpu.sync_copy(b0, shared.at[idx_s.at[j0]], add=True)

            @pl.when(i < NCHUNK // 2 - 1)
            def _():
                pltpu.async_copy(rows.at[pl.ds(base + (j0 + 2) * CH, CH)],
                                 b0, sl0)

            wait64(sl1)
            pltpu.sync_copy(b1, shared.at[idx_s.at[j0 + 1]], add=True)
            return carry

        lax.fori_loop(0, NCHUNK // 2, body, 0)
        plsc.subcore_barrier()
        pltpu.sync_copy(shared.at[pl.ds(sid * NPT, NPT)],
                        acc2.at[cid, pl.ds(sid * NPT, NPT)])

    return _sc_scatter


def _scatter(rows, src3):
    return _sc_scatter_call()(rows, src3)


def _time_body(sig, tw1t, tb1, tw2t, tb2, w1dt, b1c, out):
    s = sig[0:1, 0:1]
    i = lax.broadcasted_iota(i32, (8, TD // 2), 1).astype(f32)
    freqs = jnp.exp(i * (jnp.log(1000.0) / (TD // 2 - 1)))
    args = s * freqs
    emb = jnp.concatenate([jnp.sin(args), jnp.cos(args)], axis=1)
    te = jax.nn.silu(_dot(emb, tw1t[...]) + tb1[...])
    te = _dot(te, tw2t[...]) + tb2[...]
    out[...] = _dot(te, w1dt[...]) + b1c[...]


def _pro_body(ti, tg, nt, sp, x8,
              pet, net, let, modt, pw1r, pb1, pw2t, pb2, a0t, b0t,
              h_o, p_o, q_o):
    tiv = ti[...]
    oh25 = (tiv == lax.broadcasted_iota(i32, (BN, 32), 1)).astype(f32)
    pe = _dot(oh25, pet[...])
    ohn = (jnp.maximum(tiv, 0)
           == lax.broadcasted_iota(i32, (BN, 8), 1)).astype(f32)
    ne = _dot(ohn, net[...])
    oh64 = (tiv == lax.broadcasted_iota(i32, (BN, 64), 1)).astype(f32)
    le = _dot(oh64, let[...])
    tgv = tg[...]
    feats = jnp.where(tgv == 0, pe, 0.0)
    feats = jnp.where(tgv == 1, ne, feats)
    feats = jnp.where(tgv == 2, le, feats)
    ohm = (nt[...] == lax.broadcasted_iota(i32, (BN, 8), 1)).astype(f32)
    feats = feats + _dot(ohm, modt[...])
    p = jax.nn.silu(sp[...] * pw1r[0:1, :] + pb1[0:1, :])
    feats = feats + _dot(p, pw2t[...]) + pb2[0:1, :]
    h_o[...] = feats
    g = _dot(feats, a0t[...])
    k = _dot(feats, b0t[...])
    xs = x8[...][:, 0:3]
    z13 = jnp.zeros((BN, PADW), f32)
    p_o[...] = jnp.concatenate([g, xs, z13], axis=1)
    q_o[...] = jnp.concatenate([k, xs, z13], axis=1)


def _edge_body(pr, qr, et, w2t, cw1t, drbf, ctab, misc, out):
    p = pr[...]
    q = qr[...]
    g = p[:, :H] + q[:, :H]
    rel = p[:, H:H + 3] - q[:, H:H + 3]
    d2 = jnp.sum(rel * rel, axis=1, keepdims=True)
    dist = jnp.maximum(jnp.sqrt(d2), 1e-6)
    centers = (lax.broadcasted_iota(i32, (BE, RBF), 1).astype(f32)
               * (12.0 / (RBF - 1)))
    diff = dist - centers
    rbf = jnp.exp(-0.5 * diff * diff)
    ohe = (et[...] == lax.broadcasted_iota(i32, (BE, 8), 1)).astype(f32)
    pre = g + _dot(rbf, drbf[...]) + _dot(ohe, ctab[...]) + misc[0:1, :]
    m1 = jax.nn.silu(pre)
    msgs = jax.nn.silu(_dot(m1, w2t[...]) + misc[1:2, :])
    c1 = jax.nn.silu(_dot(msgs, cw1t[...]) + misc[2:3, :])
    cs = jnp.sum(c1 * misc[3:4, :], axis=1, keepdims=True) + misc[4:5, 0:1]
    w = cs / (dist + 1.0)
    out[...] = jnp.concatenate(
        [msgs, w * rel, jnp.zeros((BE, PADW), f32)], axis=1)


def _node_common(h, x8, acc, n1ht, n1ft, n2t, nmisc):
    a = acc[0] + acc[1]
    fa = a[:, :H]
    ca = a[:, H:H + 3]
    hv = h[...]
    t1 = jax.nn.silu(_dot(hv, n1ht[...]) + _dot(fa, n1ft[...])
                     + nmisc[0:1, :])
    hn = hv + _dot(t1, n2t[...]) + nmisc[1:2, :]
    xn = x8[...] + jnp.concatenate([ca, jnp.zeros((BN, 5), f32)], axis=1)
    return hn, xn


def _node_body(h, x8, acc, n1ht, n1ft, n2t, nmisc, a1t, b1t,
               h_o, x_o, p_o, q_o):
    hn, xn = _node_common(h, x8, acc, n1ht, n1ft, n2t, nmisc)
    h_o[...] = hn
    x_o[...] = xn
    g = _dot(hn, a1t[...])
    k = _dot(hn, b1t[...])
    xs = xn[:, 0:3]
    z13 = jnp.zeros((BN, PADW), f32)
    p_o[...] = jnp.concatenate([g, xs, z13], axis=1)
    q_o[...] = jnp.concatenate([k, xs, z13], axis=1)


def _node_final_body(h, x8, acc, n1ht, n1ft, n2t, nmisc, ow1t, ow2t8, ob8,
                     res_o):
    hn, xn = _node_common(h, x8, acc, n1ht, n1ft, n2t, nmisc)
    d1 = jax.nn.silu(_dot(hn, ow1t[...]) + nmisc[2:3, :])
    res_o[...] = xn + _dot(d1, ow2t8[...]) + ob8[0:1, :]


def _full(shape):
    return pl.BlockSpec(shape, lambda i: (0,) * len(shape))


def _rows(shape):
    return pl.BlockSpec(shape, lambda i: (i,) + (0,) * (len(shape) - 1))


_time_call = pl.pallas_call(
    _time_body,
    out_shape=jax.ShapeDtypeStruct((8, L * H), f32),
)

_pro_call = pl.pallas_call(
    _pro_body,
    grid=(NP // BN,),
    in_specs=[
        _rows((BN, 1)), _rows((BN, 1)), _rows((BN, 1)), _rows((BN, 1)),
        _rows((BN, 8)),
        _full((32, H)), _full((8, H)), _full((64, H)), _full((8, H)),
        _full((8, H)), _full((8, H)), _full((H, H)), _full((8, H)),
        _full((H, H)), _full((H, H)),
    ],
    out_specs=[_rows((BN, H)), _rows((BN, DW)), _rows((BN, DW))],
    out_shape=[jax.ShapeDtypeStruct((NP, H), f32),
               jax.ShapeDtypeStruct((NP, DW), f32),
               jax.ShapeDtypeStruct((NP, DW), f32)],
)

_edge_call = pl.pallas_call(
    _edge_body,
    grid=(EP // BE,),
    in_specs=[
        _rows((BE, DW)), _rows((BE, DW)), _rows((BE, 1)),
        _full((H, H)), _full((H, H)), _full((RBF, H)), _full((8, H)),
        _full((8, H)),
    ],
    out_specs=_rows((BE, DW)),
    out_shape=jax.ShapeDtypeStruct((EP, DW), f32),
)

_acc_spec = pl.BlockSpec((2, BN, DW), lambda i: (0, i, 0))

_node_call = pl.pallas_call(
    _node_body,
    grid=(NP // BN,),
    in_specs=[
        _rows((BN, H)), _rows((BN, 8)), _acc_spec,
        _full((H, H)), _full((H, H)), _full((H, H)), _full((8, H)),
        _full((H, H)), _full((H, H)),
    ],
    out_specs=[_rows((BN, H)), _rows((BN, 8)),
               _rows((BN, DW)), _rows((BN, DW))],
    out_shape=[jax.ShapeDtypeStruct((NP, H), f32),
               jax.ShapeDtypeStruct((NP, 8), f32),
               jax.ShapeDtypeStruct((NP, DW), f32),
               jax.ShapeDtypeStruct((NP, DW), f32)],
)

_node_final_call = pl.pallas_call(
    _node_final_body,
    grid=(NP // BN,),
    in_specs=[
        _rows((BN, H)), _rows((BN, 8)), _acc_spec,
        _full((H, H)), _full((H, H)), _full((H, H)), _full((8, H)),
        _full((H, H)), _full((H, 8)), _full((8, 8)),
    ],
    out_specs=_rows((BN, 8)),
    out_shape=jax.ShapeDtypeStruct((NP, 8), f32),
)


def kernel(noisy_coords, sigma, token_groups, token_indices, node_types,
           sequence_positions, edge_index, edge_types, params):
    layers = params['layers']

    def bc8(v):
        return jnp.broadcast_to(v[None, :], (8, v.shape[0]))

    x8 = jnp.pad(noisy_coords[0].astype(f32), ((0, NP - N), (0, 5)))
    ti2 = jnp.pad(token_indices.astype(i32), (0, NP - N))[:, None]
    tg2 = jnp.pad(token_groups.astype(i32), (0, NP - N))[:, None]
    nt2 = jnp.pad(node_types.astype(i32), (0, NP - N))[:, None]
    sp2 = jnp.pad(sequence_positions.astype(f32), (0, NP - N))[:, None]
    src = edge_index[:, 0].astype(i32)
    dst = edge_index[:, 1].astype(i32)
    idx_pad = jnp.full((EP - E,), PAD_ROW, i32)
    src3 = jnp.concatenate([src, idx_pad]).reshape(NTILES, NCHUNK, CH)
    dst3 = jnp.concatenate([dst, idx_pad]).reshape(NTILES, NCHUNK, CH)
    et2 = jnp.pad(edge_types.astype(i32), (0, EP - E))[:, None]

    w1dt = jnp.concatenate(
        [lp['edge_W1'][:, 288:320].T for lp in layers], axis=1)
    b1c = jnp.broadcast_to(
        jnp.concatenate([lp['edge_b1'] for lp in layers])[None, :],
        (8, L * H))
    sig8 = jnp.broadcast_to(sigma.reshape(1, 1).astype(f32), (8, 128))
    tv = _time_call(sig8, params['time_W1'].T, bc8(params['time_b1']),
                    params['time_W2'].T, bc8(params['time_b2']), w1dt, b1c)

    at = [lp['edge_W1'][:, 0:H].T for lp in layers]
    bt = [lp['edge_W1'][:, H:2 * H].T for lp in layers]
    drbf = [lp['edge_W1'][:, 3 * H + TD:].T for lp in layers]
    ctab = [jnp.pad(_dot(lp['edge_emb'], lp['edge_W1'][:, 2 * H:3 * H].T),
                    ((0, 3), (0, 0))) for lp in layers]
    w2t = [lp['edge_W2'].T for lp in layers]
    cw1t = [lp['coord_W1'].T for lp in layers]
    miscs = []
    for l, lp in enumerate(layers):
        miscs.append(jnp.stack([
            tv[0, l * H:(l + 1) * H],
            lp['edge_b2'],
            lp['coord_b1'],
            lp['coord_W2'][0],
            jnp.full((H,), lp['coord_b2'][0]),
            jnp.zeros((H,), f32), jnp.zeros((H,), f32), jnp.zeros((H,), f32),
        ]))
    n1ht = [lp['node_W1'][:, 0:H].T for lp in layers]
    n1ft = [lp['node_W1'][:, H:].T for lp in layers]
    n2t = [lp['node_W2'].T for lp in layers]
    nmiscs = []
    for lp in layers:
        nmiscs.append(jnp.stack([
            lp['node_b1'], lp['node_b2'], params['out_b1'],
            jnp.zeros((H,), f32), jnp.zeros((H,), f32),
            jnp.zeros((H,), f32), jnp.zeros((H,), f32), jnp.zeros((H,), f32),
        ]))
    ow1t = params['out_W1'].T
    ow2t8 = jnp.pad(params['out_W2'].T, ((0, 0), (0, 5)))
    ob8 = jnp.broadcast_to(jnp.pad(params['out_b2'], (0, 5))[None, :], (8, 8))

    h, tab_p, tab_q = _pro_call(
        ti2, tg2, nt2, sp2, x8,
        jnp.pad(params['protein_emb'], ((0, 7), (0, 0))),
        params['nucleotide_emb'],
        params['ligand_emb'],
        jnp.pad(params['modality_emb'], ((0, 5), (0, 0))),
        bc8(params['pos_W1'][:, 0]), bc8(params['pos_b1']),
        params['pos_W2'].T, bc8(params['pos_b2']),
        at[0], bt[0])

    for l in range(L):
        rows_p, rows_q = _gather(tab_p, tab_q, src3, dst3)
        rows = _edge_call(rows_p, rows_q, et2, w2t[l], cw1t[l], drbf[l],
                          ctab[l], miscs[l])
        acc2 = _scatter(rows, src3)
        if l < L - 1:
            h, x8, tab_p, tab_q = _node_call(
                h, x8, acc2, n1ht[l], n1ft[l], n2t[l], nmiscs[l],
                at[l + 1], bt[l + 1])
        else:
            res = _node_final_call(
                h, x8, acc2, n1ht[l], n1ft[l], n2t[l], nmiscs[l],
                ow1t, ow2t8, ob8)

    return res[:N, :3][None]


# speedup vs baseline: 4.5869x; 1.4849x over previous
import functools

import jax
import jax.numpy as jnp
from jax import lax
from jax.experimental import pallas as pl
from jax.experimental.pallas import tpu as pltpu
from jax.experimental.pallas import tpu_sc as plsc

N, E, H, TD, RBF, L = 10000, 160000, 96, 32, 16, 4
NP = 10240
EP = 163840
DW = 128
PADW = DW - H - 3
PAD_ROW = NP - 2
NTILES, NCHUNK, CH = 32, 40, 128
EPT = NCHUNK * CH
BE = 1280
BN = 512
NPT = NP // 16

f32 = jnp.float32
i32 = jnp.int32
_dot = functools.partial(jnp.dot, preferred_element_type=jnp.float32)


@functools.lru_cache(maxsize=None)
def _sc_mesh():
    return plsc.VectorSubcoreMesh(
        core_axis_name="c", subcore_axis_name="s",
        num_cores=2, num_subcores=16)


NCH2 = 2 * NCHUNK


@functools.lru_cache(maxsize=None)
def _sc_gather_call():
    @functools.partial(
        pl.kernel,
        out_type=jax.ShapeDtypeStruct((2, EP, DW), f32),
        mesh=_sc_mesh(),
        scratch_types=[
            pltpu.VMEM_SHARED((NP, DW), f32),
            pltpu.VMEM((NCH2, CH), i32),
            pltpu.VMEM((CH, DW), f32),
            pltpu.VMEM((CH, DW), f32),
        ] + [pltpu.SemaphoreType.DMA] * 4,
    )
    def _sc_gather(tabs, sd3, out_pq, shared, idx_v, b0, b1,
                   sg0, sg1, ss0, ss1):
        cid = lax.axis_index("c")
        sid = lax.axis_index("s")
        base = sid * (NCH2 * CH)
        pltpu.sync_copy(tabs.at[cid, pl.ds(sid * NPT, NPT)],
                        shared.at[pl.ds(sid * NPT, NPT)])
        pltpu.sync_copy(sd3.at[cid, sid], idx_v)
        plsc.subcore_barrier()

        def wait64(sem):
            pltpu.make_async_copy(out_pq.at[0, pl.ds(0, CH)], b0, sem).wait()

        pltpu.async_copy(shared.at[idx_v.at[0]], b0, sg0)

        def body(i, carry):
            j0 = 2 * i

            @pl.when(i > 0)
            def _():
                wait64(ss1)

            pltpu.async_copy(shared.at[idx_v.at[j0 + 1]], b1, sg1)
            wait64(sg0)
            pltpu.async_copy(b0, out_pq.at[cid, pl.ds(base + j0 * CH, CH)],
                             ss0)

            @pl.when(i < NCH2 // 2 - 1)
            def _():
                wait64(ss0)
                pltpu.async_copy(shared.at[idx_v.at[j0 + 2]], b0, sg0)

            wait64(sg1)
            pltpu.async_copy(b1,
                             out_pq.at[cid, pl.ds(base + (j0 + 1) * CH, CH)],
                             ss1)
            return carry

        lax.fori_loop(0, NCH2 // 2, body, 0)
        wait64(ss0)
        wait64(ss1)

    return _sc_gather


def _gather(tabs, sd3):
    return _sc_gather_call()(tabs, sd3)


@functools.lru_cache(maxsize=None)
def _sc_scatter_call():
    @functools.partial(
        pl.kernel,
        out_type=jax.ShapeDtypeStruct((2, NP, DW), f32),
        mesh=_sc_mesh(),
        scratch_types=[
            pltpu.VMEM_SHARED((NP, DW), f32),
            pltpu.VMEM((NCHUNK, CH), i32),
            pltpu.VMEM((CH, DW), f32),
            pltpu.VMEM((CH, DW), f32),
            pltpu.SemaphoreType.DMA,
            pltpu.SemaphoreType.DMA,
        ],
    )
    def _sc_scatter(rows, src3, acc2, shared, idx_s, b0, b1, sl0, sl1):
        cid = lax.axis_index("c")
        sid = lax.axis_index("s")
        wid = cid * 16 + sid
        base = wid * EPT

        def zrow(r, carry):
            for c in range(DW // 16):
                b0[r, pl.ds(c * 16, 16)] = jnp.zeros((16,), f32)
            return carry

        lax.fori_loop(0, CH, zrow, 0)
        for c in range(NPT // CH):
            pltpu.sync_copy(b0, shared.at[pl.ds(sid * NPT + c * CH, CH)])
        plsc.subcore_barrier()

        pltpu.sync_copy(src3.at[wid], idx_s)

        def wait64(sem):
            pltpu.make_async_copy(rows.at[pl.ds(0, CH)], b0, sem).wait()

        pltpu.async_copy(rows.at[pl.ds(base, CH)], b0, sl0)

        def body(i, carry):
            j0 = 2 * i
            pltpu.async_copy(rows.at[pl.ds(base + (j0 + 1) * CH, CH)], b1,
                             sl1)
            wait64(sl0)
            pltpu.sync_copy(b0, shared.at[idx_s.at[j0]], add=True)

            @pl.when(i < NCHUNK // 2 - 1)
            def _():
                pltpu.async_copy(rows.at[pl.ds(base + (j0 + 2) * CH, CH)],
                                 b0, sl0)

            wait64(sl1)
            pltpu.sync_copy(b1, shared.at[idx_s.at[j0 + 1]], add=True)
            return carry

        lax.fori_loop(0, NCHUNK // 2, body, 0)
        plsc.subcore_barrier()
        pltpu.sync_copy(shared.at[pl.ds(sid * NPT, NPT)],
                        acc2.at[cid, pl.ds(sid * NPT, NPT)])

    return _sc_scatter


def _scatter(rows, src3):
    return _sc_scatter_call()(rows, src3)


def _time_body(sig, tw1t, tb1, tw2t, tb2, w1dt, b1c, out):
    s = sig[0:1, 0:1]
    i = lax.broadcasted_iota(i32, (8, TD // 2), 1).astype(f32)
    freqs = jnp.exp(i * (jnp.log(1000.0) / (TD // 2 - 1)))
    args = s * freqs
    emb = jnp.concatenate([jnp.sin(args), jnp.cos(args)], axis=1)
    te = jax.nn.silu(_dot(emb, tw1t[...]) + tb1[...])
    te = _dot(te, tw2t[...]) + tb2[...]
    out[...] = _dot(te, w1dt[...]) + b1c[...]


def _pro_body(ti, tg, nt, sp, x8,
              pet, net, let, modt, pw1r, pb1, pw2t, pb2, a0t, b0t,
              h_o, t_o):
    tiv = ti[...]
    oh25 = (tiv == lax.broadcasted_iota(i32, (BN, 32), 1)).astype(f32)
    pe = _dot(oh25, pet[...])
    ohn = (jnp.maximum(tiv, 0)
           == lax.broadcasted_iota(i32, (BN, 8), 1)).astype(f32)
    ne = _dot(ohn, net[...])
    oh64 = (tiv == lax.broadcasted_iota(i32, (BN, 64), 1)).astype(f32)
    le = _dot(oh64, let[...])
    tgv = tg[...]
    feats = jnp.where(tgv == 0, pe, 0.0)
    feats = jnp.where(tgv == 1, ne, feats)
    feats = jnp.where(tgv == 2, le, feats)
    ohm = (nt[...] == lax.broadcasted_iota(i32, (BN, 8), 1)).astype(f32)
    feats = feats + _dot(ohm, modt[...])
    p = jax.nn.silu(sp[...] * pw1r[0:1, :] + pb1[0:1, :])
    feats = feats + _dot(p, pw2t[...]) + pb2[0:1, :]
    h_o[...] = feats
    g = _dot(feats, a0t[...])
    k = _dot(feats, b0t[...])
    xs = x8[...][:, 0:3]
    z13 = jnp.zeros((BN, PADW), f32)
    t_o[0] = jnp.concatenate([g, xs, z13], axis=1)
    t_o[1] = jnp.concatenate([k, xs, z13], axis=1)


def _edge_body(pr, qr, et, w2t, cw1t, drbf, ctab, misc, out):
    p = pr[0]
    q = qr[0]
    g = p[:, :H] + q[:, :H]
    rel = p[:, H:H + 3] - q[:, H:H + 3]
    d2 = jnp.sum(rel * rel, axis=1, keepdims=True)
    dist = jnp.maximum(jnp.sqrt(d2), 1e-6)
    centers = (lax.broadcasted_iota(i32, (BE, RBF), 1).astype(f32)
               * (12.0 / (RBF - 1)))
    diff = dist - centers
    rbf = jnp.exp(-0.5 * diff * diff)
    ohe = (et[...] == lax.broadcasted_iota(i32, (BE, 8), 1)).astype(f32)
    pre = g + _dot(rbf, drbf[...]) + _dot(ohe, ctab[...]) + misc[0:1, :]
    m1 = jax.nn.silu(pre)
    msgs = jax.nn.silu(_dot(m1, w2t[...]) + misc[1:2, :])
    c1 = jax.nn.silu(_dot(msgs, cw1t[...]) + misc[2:3, :])
    cs = jnp.sum(c1 * misc[3:4, :], axis=1, keepdims=True) + misc[4:5, 0:1]
    w = cs / (dist + 1.0)
    out[...] = jnp.concatenate(
        [msgs, w * rel, jnp.zeros((BE, PADW), f32)], axis=1)


def _node_common(h, x8, acc, n1ht, n1ft, n2t, nmisc):
    a = acc[0] + acc[1]
    fa = a[:, :H]
    ca = a[:, H:H + 3]
    hv = h[...]
    t1 = jax.nn.silu(_dot(hv, n1ht[...]) + _dot(fa, n1ft[...])
                     + nmisc[0:1, :])
    hn = hv + _dot(t1, n2t[...]) + nmisc[1:2, :]
    xn = x8[...] + jnp.concatenate([ca, jnp.zeros((BN, 5), f32)], axis=1)
    return hn, xn


def _node_body(h, x8, acc, n1ht, n1ft, n2t, nmisc, a1t, b1t,
               h_o, x_o, t_o):
    hn, xn = _node_common(h, x8, acc, n1ht, n1ft, n2t, nmisc)
    h_o[...] = hn
    x_o[...] = xn
    g = _dot(hn, a1t[...])
    k = _dot(hn, b1t[...])
    xs = xn[:, 0:3]
    z13 = jnp.zeros((BN, PADW), f32)
    t_o[0] = jnp.concatenate([g, xs, z13], axis=1)
    t_o[1] = jnp.concatenate([k, xs, z13], axis=1)


def _node_final_body(h, x8, acc, n1ht, n1ft, n2t, nmisc, ow1t, ow2t8, ob8,
                     res_o):
    hn, xn = _node_common(h, x8, acc, n1ht, n1ft, n2t, nmisc)
    d1 = jax.nn.silu(_dot(hn, ow1t[...]) + nmisc[2:3, :])
    res_o[...] = xn + _dot(d1, ow2t8[...]) + ob8[0:1, :]


def _full(shape):
    return pl.BlockSpec(shape, lambda i: (0,) * len(shape))


def _rows(shape):
    return pl.BlockSpec(shape, lambda i: (i,) + (0,) * (len(shape) - 1))


_time_call = pl.pallas_call(
    _time_body,
    out_shape=jax.ShapeDtypeStruct((8, L * H), f32),
)

_pro_call = pl.pallas_call(
    _pro_body,
    grid=(NP // BN,),
    in_specs=[
        _rows((BN, 1)), _rows((BN, 1)), _rows((BN, 1)), _rows((BN, 1)),
        _rows((BN, 8)),
        _full((32, H)), _full((8, H)), _full((64, H)), _full((8, H)),
        _full((8, H)), _full((8, H)), _full((H, H)), _full((8, H)),
        _full((H, H)), _full((H, H)),
    ],
    out_specs=[_rows((BN, H)),
               pl.BlockSpec((2, BN, DW), lambda i: (0, i, 0))],
    out_shape=[jax.ShapeDtypeStruct((NP, H), f32),
               jax.ShapeDtypeStruct((2, NP, DW), f32)],
)

_edge_call = pl.pallas_call(
    _edge_body,
    grid=(EP // BE,),
    in_specs=[
        pl.BlockSpec((1, BE, DW), lambda i: (0, i, 0)),
        pl.BlockSpec((1, BE, DW), lambda i: (1, i, 0)),
        _rows((BE, 1)),
        _full((H, H)), _full((H, H)), _full((RBF, H)), _full((8, H)),
        _full((8, H)),
    ],
    out_specs=_rows((BE, DW)),
    out_shape=jax.ShapeDtypeStruct((EP, DW), f32),
)

_acc_spec = pl.BlockSpec((2, BN, DW), lambda i: (0, i, 0))

_node_call = pl.pallas_call(
    _node_body,
    grid=(NP // BN,),
    in_specs=[
        _rows((BN, H)), _rows((BN, 8)), _acc_spec,
        _full((H, H)), _full((H, H)), _full((H, H)), _full((8, H)),
        _full((H, H)), _full((H, H)),
    ],
    out_specs=[_rows((BN, H)), _rows((BN, 8)),
               pl.BlockSpec((2, BN, DW), lambda i: (0, i, 0))],
    out_shape=[jax.ShapeDtypeStruct((NP, H), f32),
               jax.ShapeDtypeStruct((NP, 8), f32),
               jax.ShapeDtypeStruct((2, NP, DW), f32)],
)

_node_final_call = pl.pallas_call(
    _node_final_body,
    grid=(NP // BN,),
    in_specs=[
        _rows((BN, H)), _rows((BN, 8)), _acc_spec,
        _full((H, H)), _full((H, H)), _full((H, H)), _full((8, H)),
        _full((H, H)), _full((H, 8)), _full((8, 8)),
    ],
    out_specs=_rows((BN, 8)),
    out_shape=jax.ShapeDtypeStruct((NP, 8), f32),
)


def kernel(noisy_coords, sigma, token_groups, token_indices, node_types,
           sequence_positions, edge_index, edge_types, params):
    layers = params['layers']

    def bc8(v):
        return jnp.broadcast_to(v[None, :], (8, v.shape[0]))

    x8 = jnp.pad(noisy_coords[0].astype(f32), ((0, NP - N), (0, 5)))
    ti2 = jnp.pad(token_indices.astype(i32), (0, NP - N))[:, None]
    tg2 = jnp.pad(token_groups.astype(i32), (0, NP - N))[:, None]
    nt2 = jnp.pad(node_types.astype(i32), (0, NP - N))[:, None]
    sp2 = jnp.pad(sequence_positions.astype(f32), (0, NP - N))[:, None]
    src = edge_index[:, 0].astype(i32)
    dst = edge_index[:, 1].astype(i32)
    idx_pad = jnp.full((EP - E,), PAD_ROW, i32)
    src_p = jnp.concatenate([src, idx_pad])
    dst_p = jnp.concatenate([dst, idx_pad])
    src3 = src_p.reshape(NTILES, NCHUNK, CH)
    sd3 = jnp.stack([src_p.reshape(16, NCH2, CH),
                     dst_p.reshape(16, NCH2, CH)])
    et2 = jnp.pad(edge_types.astype(i32), (0, EP - E))[:, None]

    w1dt = jnp.concatenate(
        [lp['edge_W1'][:, 288:320].T for lp in layers], axis=1)
    b1c = jnp.broadcast_to(
        jnp.concatenate([lp['edge_b1'] for lp in layers])[None, :],
        (8, L * H))
    sig8 = jnp.broadcast_to(sigma.reshape(1, 1).astype(f32), (8, 128))
    tv = _time_call(sig8, params['time_W1'].T, bc8(params['time_b1']),
                    params['time_W2'].T, bc8(params['time_b2']), w1dt, b1c)

    at = [lp['edge_W1'][:, 0:H].T for lp in layers]
    bt = [lp['edge_W1'][:, H:2 * H].T for lp in layers]
    drbf = [lp['edge_W1'][:, 3 * H + TD:].T for lp in layers]
    ctab = [jnp.pad(_dot(lp['edge_emb'], lp['edge_W1'][:, 2 * H:3 * H].T),
                    ((0, 3), (0, 0))) for lp in layers]
    w2t = [lp['edge_W2'].T for lp in layers]
    cw1t = [lp['coord_W1'].T for lp in layers]
    miscs = []
    for l, lp in enumerate(layers):
        miscs.append(jnp.stack([
            tv[0, l * H:(l + 1) * H],
            lp['edge_b2'],
            lp['coord_b1'],
            lp['coord_W2'][0],
            jnp.full((H,), lp['coord_b2'][0]),
            jnp.zeros((H,), f32), jnp.zeros((H,), f32), jnp.zeros((H,), f32),
        ]))
    n1ht = [lp['node_W1'][:, 0:H].T for lp in layers]
    n1ft = [lp['node_W1'][:, H:].T for lp in layers]
    n2t = [lp['node_W2'].T for lp in layers]
    nmiscs = []
    for lp in layers:
        nmiscs.append(jnp.stack([
            lp['node_b1'], lp['node_b2'], params['out_b1'],
            jnp.zeros((H,), f32), jnp.zeros((H,), f32),
            jnp.zeros((H,), f32), jnp.zeros((H,), f32), jnp.zeros((H,), f32),
        ]))
    ow1t = params['out_W1'].T
    ow2t8 = jnp.pad(params['out_W2'].T, ((0, 0), (0, 5)))
    ob8 = jnp.broadcast_to(jnp.pad(params['out_b2'], (0, 5))[None, :], (8, 8))

    h, tabs = _pro_call(
        ti2, tg2, nt2, sp2, x8,
        jnp.pad(params['protein_emb'], ((0, 7), (0, 0))),
        params['nucleotide_emb'],
        params['ligand_emb'],
        jnp.pad(params['modality_emb'], ((0, 5), (0, 0))),
        bc8(params['pos_W1'][:, 0]), bc8(params['pos_b1']),
        params['pos_W2'].T, bc8(params['pos_b2']),
        at[0], bt[0])

    for l in range(L):
        rows_pq = _gather(tabs, sd3)
        rows = _edge_call(rows_pq, rows_pq, et2, w2t[l], cw1t[l], drbf[l],
                          ctab[l], miscs[l])
        acc2 = _scatter(rows, src3)
        if l < L - 1:
            h, x8, tabs = _node_call(
                h, x8, acc2, n1ht[l], n1ft[l], n2t[l], nmiscs[l],
                at[l + 1], bt[l + 1])
        else:
            res = _node_final_call(
                h, x8, acc2, n1ht[l], n1ft[l], n2t[l], nmiscs[l],
                ow1t, ow2t8, ob8)

    return res[:N, :3][None]


# speedup vs baseline: 4.9363x; 1.0762x over previous
import functools

import jax
import jax.numpy as jnp
from jax import lax
from jax.experimental import pallas as pl
from jax.experimental.pallas import tpu as pltpu
from jax.experimental.pallas import tpu_sc as plsc

N, E, H, TD, RBF, L = 10000, 160000, 96, 32, 16, 4
NP = 10240
EP = 163840
DW = 128
PADW = DW - H - 3
PAD_ROW = NP - 2
NTILES, NCHUNK, CH = 32, 40, 128
EPT = NCHUNK * CH
BE = 1280
BN = 512
NPT = NP // 16

f32 = jnp.float32
i32 = jnp.int32
_dot = functools.partial(jnp.dot, preferred_element_type=jnp.float32)


@functools.lru_cache(maxsize=None)
def _sc_mesh():
    return plsc.VectorSubcoreMesh(
        core_axis_name="c", subcore_axis_name="s",
        num_cores=2, num_subcores=16)


EP2 = EP // 2
NCH2 = 40
NCHS = 20


@functools.lru_cache(maxsize=None)
def _sc_gather_call():
    @functools.partial(
        pl.kernel,
        out_type=jax.ShapeDtypeStruct((2, EP2, DW), f32),
        mesh=_sc_mesh(),
        scratch_types=[
            pltpu.VMEM_SHARED((NP, DW), f32),
            pltpu.VMEM((NCH2, CH), i32),
            pltpu.VMEM((CH, DW), f32),
            pltpu.VMEM((CH, DW), f32),
        ] + [pltpu.SemaphoreType.DMA] * 4,
    )
    def _sc_gather(tabs, sd3, out_pq, shared, idx_v, b0, b1,
                   sg0, sg1, ss0, ss1):
        cid = lax.axis_index("c")
        sid = lax.axis_index("s")
        base = sid * (NCH2 * CH)
        pltpu.sync_copy(tabs.at[cid, pl.ds(sid * NPT, NPT)],
                        shared.at[pl.ds(sid * NPT, NPT)])
        pltpu.sync_copy(sd3.at[cid, sid], idx_v)
        plsc.subcore_barrier()

        def wait64(sem):
            pltpu.make_async_copy(out_pq.at[0, pl.ds(0, CH)], b0, sem).wait()

        pltpu.async_copy(shared.at[idx_v.at[0]], b0, sg0)

        def body(i, carry):
            j0 = 2 * i

            @pl.when(i > 0)
            def _():
                wait64(ss1)

            pltpu.async_copy(shared.at[idx_v.at[j0 + 1]], b1, sg1)
            wait64(sg0)
            pltpu.async_copy(b0, out_pq.at[cid, pl.ds(base + j0 * CH, CH)],
                             ss0)

            @pl.when(i < NCH2 // 2 - 1)
            def _():
                wait64(ss0)
                pltpu.async_copy(shared.at[idx_v.at[j0 + 2]], b0, sg0)

            wait64(sg1)
            pltpu.async_copy(b1,
                             out_pq.at[cid, pl.ds(base + (j0 + 1) * CH, CH)],
                             ss1)
            return carry

        lax.fori_loop(0, NCH2 // 2, body, 0)
        wait64(ss0)
        wait64(ss1)

    return _sc_gather


def _gather(tabs, sd3):
    return _sc_gather_call()(tabs, sd3)


@functools.lru_cache(maxsize=None)
def _sc_scatter_call():
    @functools.partial(
        pl.kernel,
        out_type=jax.ShapeDtypeStruct((2, NP, DW), f32),
        mesh=_sc_mesh(),
        scratch_types=[
            pltpu.VMEM_SHARED((NP, DW), f32),
            pltpu.VMEM((NCHS, CH), i32),
            pltpu.VMEM((CH, DW), f32),
            pltpu.VMEM((CH, DW), f32),
            pltpu.SemaphoreType.DMA,
            pltpu.SemaphoreType.DMA,
        ],
    )
    def _sc_scatter(rows, src3, acc2, shared, idx_s, b0, b1, sl0, sl1):
        cid = lax.axis_index("c")
        sid = lax.axis_index("s")
        wid = cid * 16 + sid
        base = wid * (NCHS * CH)

        def zrow(r, carry):
            for c in range(DW // 16):
                b0[r, pl.ds(c * 16, 16)] = jnp.zeros((16,), f32)
            return carry

        lax.fori_loop(0, CH, zrow, 0)
        for c in range(NPT // CH):
            pltpu.sync_copy(b0, shared.at[pl.ds(sid * NPT + c * CH, CH)])
        plsc.subcore_barrier()

        pltpu.sync_copy(src3.at[wid], idx_s)

        def wait64(sem):
            pltpu.make_async_copy(rows.at[pl.ds(0, CH)], b0, sem).wait()

        pltpu.async_copy(rows.at[pl.ds(base, CH)], b0, sl0)

        def body(i, carry):
            j0 = 2 * i
            pltpu.async_copy(rows.at[pl.ds(base + (j0 + 1) * CH, CH)], b1,
                             sl1)
            wait64(sl0)
            pltpu.sync_copy(b0, shared.at[idx_s.at[j0]], add=True)

            @pl.when(i < NCHS // 2 - 1)
            def _():
                pltpu.async_copy(rows.at[pl.ds(base + (j0 + 2) * CH, CH)],
                                 b0, sl0)

            wait64(sl1)
            pltpu.sync_copy(b1, shared.at[idx_s.at[j0 + 1]], add=True)
            return carry

        lax.fori_loop(0, NCHS // 2, body, 0)
        plsc.subcore_barrier()
        pltpu.sync_copy(shared.at[pl.ds(sid * NPT, NPT)],
                        acc2.at[cid, pl.ds(sid * NPT, NPT)])

    return _sc_scatter


def _scatter(rows, src3):
    return _sc_scatter_call()(rows, src3)


def _time_body(sig, tw1t, tb1, tw2t, tb2, w1dt, b1c, out):
    s = sig[0:1, 0:1]
    i = lax.broadcasted_iota(i32, (8, TD // 2), 1).astype(f32)
    freqs = jnp.exp(i * (jnp.log(1000.0) / (TD // 2 - 1)))
    args = s * freqs
    emb = jnp.concatenate([jnp.sin(args), jnp.cos(args)], axis=1)
    te = jax.nn.silu(_dot(emb, tw1t[...]) + tb1[...])
    te = _dot(te, tw2t[...]) + tb2[...]
    out[...] = _dot(te, w1dt[...]) + b1c[...]


def _pro_body(ti, tg, nt, sp, x8,
              pet, net, let, modt, pw1r, pb1, pw2t, pb2, a0t, b0t,
              h_o, t_o):
    tiv = ti[...]
    oh25 = (tiv == lax.broadcasted_iota(i32, (BN, 32), 1)).astype(f32)
    pe = _dot(oh25, pet[...])
    ohn = (jnp.maximum(tiv, 0)
           == lax.broadcasted_iota(i32, (BN, 8), 1)).astype(f32)
    ne = _dot(ohn, net[...])
    oh64 = (tiv == lax.broadcasted_iota(i32, (BN, 64), 1)).astype(f32)
    le = _dot(oh64, let[...])
    tgv = tg[...]
    feats = jnp.where(tgv == 0, pe, 0.0)
    feats = jnp.where(tgv == 1, ne, feats)
    feats = jnp.where(tgv == 2, le, feats)
    ohm = (nt[...] == lax.broadcasted_iota(i32, (BN, 8), 1)).astype(f32)
    feats = feats + _dot(ohm, modt[...])
    p = jax.nn.silu(sp[...] * pw1r[0:1, :] + pb1[0:1, :])
    feats = feats + _dot(p, pw2t[...]) + pb2[0:1, :]
    h_o[...] = feats
    g = _dot(feats, a0t[...])
    k = _dot(feats, b0t[...])
    xs = x8[...][:, 0:3]
    z13 = jnp.zeros((BN, PADW), f32)
    t_o[0] = jnp.concatenate([g, xs, z13], axis=1)
    t_o[1] = jnp.concatenate([k, xs, z13], axis=1)


def _edge_body(pr, qr, et, w2t, cw1t, drbf, ctab, misc, out):
    p = pr[0]
    q = qr[0]
    g = p[:, :H] + q[:, :H]
    rel = p[:, H:H + 3] - q[:, H:H + 3]
    d2 = jnp.sum(rel * rel, axis=1, keepdims=True)
    dist = jnp.maximum(jnp.sqrt(d2), 1e-6)
    centers = (lax.broadcasted_iota(i32, (BE, RBF), 1).astype(f32)
               * (12.0 / (RBF - 1)))
    diff = dist - centers
    rbf = jnp.exp(-0.5 * diff * diff)
    ohe = (et[...] == lax.broadcasted_iota(i32, (BE, 8), 1)).astype(f32)
    pre = g + _dot(rbf, drbf[...]) + _dot(ohe, ctab[...]) + misc[0:1, :]
    m1 = jax.nn.silu(pre)
    msgs = jax.nn.silu(_dot(m1, w2t[...]) + misc[1:2, :])
    c1 = jax.nn.silu(_dot(msgs, cw1t[...]) + misc[2:3, :])
    cs = jnp.sum(c1 * misc[3:4, :], axis=1, keepdims=True) + misc[4:5, 0:1]
    w = cs / (dist + 1.0)
    out[...] = jnp.concatenate(
        [msgs, w * rel, jnp.zeros((BE, PADW), f32)], axis=1)


def _node_common(h, x8, acca, accb, n1ht, n1ft, n2t, nmisc):
    a = (acca[0] + acca[1]) + (accb[0] + accb[1])
    fa = a[:, :H]
    ca = a[:, H:H + 3]
    hv = h[...]
    t1 = jax.nn.silu(_dot(hv, n1ht[...]) + _dot(fa, n1ft[...])
                     + nmisc[0:1, :])
    hn = hv + _dot(t1, n2t[...]) + nmisc[1:2, :]
    xn = x8[...] + jnp.concatenate([ca, jnp.zeros((BN, 5), f32)], axis=1)
    return hn, xn


def _node_body(h, x8, acca, accb, n1ht, n1ft, n2t, nmisc, a1t, b1t,
               h_o, x_o, t_o):
    hn, xn = _node_common(h, x8, acca, accb, n1ht, n1ft, n2t, nmisc)
    h_o[...] = hn
    x_o[...] = xn
    g = _dot(hn, a1t[...])
    k = _dot(hn, b1t[...])
    xs = xn[:, 0:3]
    z13 = jnp.zeros((BN, PADW), f32)
    t_o[0] = jnp.concatenate([g, xs, z13], axis=1)
    t_o[1] = jnp.concatenate([k, xs, z13], axis=1)


def _node_final_body(h, x8, acca, accb, n1ht, n1ft, n2t, nmisc, ow1t,
                     ow2t8, ob8, res_o):
    hn, xn = _node_common(h, x8, acca, accb, n1ht, n1ft, n2t, nmisc)
    d1 = jax.nn.silu(_dot(hn, ow1t[...]) + nmisc[2:3, :])
    res_o[...] = xn + _dot(d1, ow2t8[...]) + ob8[0:1, :]


def _full(shape):
    return pl.BlockSpec(shape, lambda i: (0,) * len(shape))


def _rows(shape):
    return pl.BlockSpec(shape, lambda i: (i,) + (0,) * (len(shape) - 1))


_time_call = pl.pallas_call(
    _time_body,
    out_shape=jax.ShapeDtypeStruct((8, L * H), f32),
)

_pro_call = pl.pallas_call(
    _pro_body,
    grid=(NP // BN,),
    in_specs=[
        _rows((BN, 1)), _rows((BN, 1)), _rows((BN, 1)), _rows((BN, 1)),
        _rows((BN, 8)),
        _full((32, H)), _full((8, H)), _full((64, H)), _full((8, H)),
        _full((8, H)), _full((8, H)), _full((H, H)), _full((8, H)),
        _full((H, H)), _full((H, H)),
    ],
    out_specs=[_rows((BN, H)),
               pl.BlockSpec((2, BN, DW), lambda i: (0, i, 0))],
    out_shape=[jax.ShapeDtypeStruct((NP, H), f32),
               jax.ShapeDtypeStruct((2, NP, DW), f32)],
)

_edge_call = pl.pallas_call(
    _edge_body,
    grid=(EP2 // BE,),
    in_specs=[
        pl.BlockSpec((1, BE, DW), lambda i: (0, i, 0)),
        pl.BlockSpec((1, BE, DW), lambda i: (1, i, 0)),
        _rows((BE, 1)),
        _full((H, H)), _full((H, H)), _full((RBF, H)), _full((8, H)),
        _full((8, H)),
    ],
    out_specs=_rows((BE, DW)),
    out_shape=jax.ShapeDtypeStruct((EP2, DW), f32),
)

_acc_spec = pl.BlockSpec((2, BN, DW), lambda i: (0, i, 0))

_node_call = pl.pallas_call(
    _node_body,
    grid=(NP // BN,),
    in_specs=[
        _rows((BN, H)), _rows((BN, 8)), _acc_spec, _acc_spec,
        _full((H, H)), _full((H, H)), _full((H, H)), _full((8, H)),
        _full((H, H)), _full((H, H)),
    ],
    out_specs=[_rows((BN, H)), _rows((BN, 8)),
               pl.BlockSpec((2, BN, DW), lambda i: (0, i, 0))],
    out_shape=[jax.ShapeDtypeStruct((NP, H), f32),
               jax.ShapeDtypeStruct((NP, 8), f32),
               jax.ShapeDtypeStruct((2, NP, DW), f32)],
)

_node_final_call = pl.pallas_call(
    _node_final_body,
    grid=(NP // BN,),
    in_specs=[
        _rows((BN, H)), _rows((BN, 8)), _acc_spec, _acc_spec,
        _full((H, H)), _full((H, H)), _full((H, H)), _full((8, H)),
        _full((H, H)), _full((H, 8)), _full((8, 8)),
    ],
    out_specs=_rows((BN, 8)),
    out_shape=jax.ShapeDtypeStruct((NP, 8), f32),
)


def kernel(noisy_coords, sigma, token_groups, token_indices, node_types,
           sequence_positions, edge_index, edge_types, params):
    layers = params['layers']

    def bc8(v):
        return jnp.broadcast_to(v[None, :], (8, v.shape[0]))

    x8 = jnp.pad(noisy_coords[0].astype(f32), ((0, NP - N), (0, 5)))
    ti2 = jnp.pad(token_indices.astype(i32), (0, NP - N))[:, None]
    tg2 = jnp.pad(token_groups.astype(i32), (0, NP - N))[:, None]
    nt2 = jnp.pad(node_types.astype(i32), (0, NP - N))[:, None]
    sp2 = jnp.pad(sequence_positions.astype(f32), (0, NP - N))[:, None]
    src = edge_index[:, 0].astype(i32)
    dst = edge_index[:, 1].astype(i32)
    idx_pad = jnp.full((EP - E,), PAD_ROW, i32)
    src_p = jnp.concatenate([src, idx_pad])
    dst_p = jnp.concatenate([dst, idx_pad])
    srcA3 = src_p[:EP2].reshape(NTILES, NCHS, CH)
    srcB3 = src_p[EP2:].reshape(NTILES, NCHS, CH)
    sdA = jnp.stack([src_p[:EP2].reshape(16, NCH2, CH),
                     dst_p[:EP2].reshape(16, NCH2, CH)])
    sdB = jnp.stack([src_p[EP2:].reshape(16, NCH2, CH),
                     dst_p[EP2:].reshape(16, NCH2, CH)])
    et_p = jnp.pad(edge_types.astype(i32), (0, EP - E))[:, None]
    etA = et_p[:EP2]
    etB = et_p[EP2:]

    w1dt = jnp.concatenate(
        [lp['edge_W1'][:, 288:320].T for lp in layers], axis=1)
    b1c = jnp.broadcast_to(
        jnp.concatenate([lp['edge_b1'] for lp in layers])[None, :],
        (8, L * H))
    sig8 = jnp.broadcast_to(sigma.reshape(1, 1).astype(f32), (8, 128))
    tv = _time_call(sig8, params['time_W1'].T, bc8(params['time_b1']),
                    params['time_W2'].T, bc8(params['time_b2']), w1dt, b1c)

    at = [lp['edge_W1'][:, 0:H].T for lp in layers]
    bt = [lp['edge_W1'][:, H:2 * H].T for lp in layers]
    drbf = [lp['edge_W1'][:, 3 * H + TD:].T for lp in layers]
    ctab = [jnp.pad(_dot(lp['edge_emb'], lp['edge_W1'][:, 2 * H:3 * H].T),
                    ((0, 3), (0, 0))) for lp in layers]
    w2t = [lp['edge_W2'].T for lp in layers]
    cw1t = [lp['coord_W1'].T for lp in layers]
    miscs = []
    for l, lp in enumerate(layers):
        miscs.append(jnp.stack([
            tv[0, l * H:(l + 1) * H],
            lp['edge_b2'],
            lp['coord_b1'],
            lp['coord_W2'][0],
            jnp.full((H,), lp['coord_b2'][0]),
            jnp.zeros((H,), f32), jnp.zeros((H,), f32), jnp.zeros((H,), f32),
        ]))
    n1ht = [lp['node_W1'][:, 0:H].T for lp in layers]
    n1ft = [lp['node_W1'][:, H:].T for lp in layers]
    n2t = [lp['node_W2'].T for lp in layers]
    nmiscs = []
    for lp in layers:
        nmiscs.append(jnp.stack([
            lp['node_b1'], lp['node_b2'], params['out_b1'],
            jnp.zeros((H,), f32), jnp.zeros((H,), f32),
            jnp.zeros((H,), f32), jnp.zeros((H,), f32), jnp.zeros((H,), f32),
        ]))
    ow1t = params['out_W1'].T
    ow2t8 = jnp.pad(params['out_W2'].T, ((0, 0), (0, 5)))
    ob8 = jnp.broadcast_to(jnp.pad(params['out_b2'], (0, 5))[None, :], (8, 8))

    h, tabs = _pro_call(
        ti2, tg2, nt2, sp2, x8,
        jnp.pad(params['protein_emb'], ((0, 7), (0, 0))),
        params['nucleotide_emb'],
        params['ligand_emb'],
        jnp.pad(params['modality_emb'], ((0, 5), (0, 0))),
        bc8(params['pos_W1'][:, 0]), bc8(params['pos_b1']),
        params['pos_W2'].T, bc8(params['pos_b2']),
        at[0], bt[0])

    for l in range(L):
        rowsA = _gather(tabs, sdA)
        rowsB = _gather(tabs, sdB)
        mA = _edge_call(rowsA, rowsA, etA, w2t[l], cw1t[l], drbf[l],
                        ctab[l], miscs[l])
        mB = _edge_call(rowsB, rowsB, etB, w2t[l], cw1t[l], drbf[l],
                        ctab[l], miscs[l])
        accA = _scatter(mA, srcA3)
        accB = _scatter(mB, srcB3)
        if l < L - 1:
            h, x8, tabs = _node_call(
                h, x8, accA, accB, n1ht[l], n1ft[l], n2t[l], nmiscs[l],
                at[l + 1], bt[l + 1])
        else:
            res = _node_final_call(
                h, x8, accA, accB, n1ht[l], n1ft[l], n2t[l], nmiscs[l],
                ow1t, ow2t8, ob8)

    return res[:N, :3][None]


# speedup vs baseline: 6.4856x; 1.3139x over previous
import functools

import jax
import jax.numpy as jnp
from jax import lax
from jax.experimental import pallas as pl
from jax.experimental.pallas import tpu as pltpu
from jax.experimental.pallas import tpu_sc as plsc

N, E, H, TD, RBF, L = 10000, 160000, 96, 32, 16, 4
NP = 10240
EP = 163840
DW = 128
PADW = DW - H - 3
PAD_ROW = NP - 2
NTILES, NCHUNK, CH = 32, 40, 128
EPT = NCHUNK * CH
BE = 2560
BN = 512
NPT = NP // 16

f32 = jnp.float32
i32 = jnp.int32
_dot = functools.partial(jnp.dot, preferred_element_type=jnp.float32)


@functools.lru_cache(maxsize=None)
def _sc_mesh():
    return plsc.VectorSubcoreMesh(
        core_axis_name="c", subcore_axis_name="s",
        num_cores=2, num_subcores=16)


EP2 = EP // 2
NCH2 = 40
NCHS = 20


@functools.lru_cache(maxsize=None)
def _sc_gather_call():
    @functools.partial(
        pl.kernel,
        out_type=jax.ShapeDtypeStruct((2, EP2, DW), f32),
        mesh=_sc_mesh(),
        scratch_types=[
            pltpu.VMEM_SHARED((NP, DW), f32),
            pltpu.VMEM((NCH2, CH), i32),
            pltpu.VMEM((CH, DW), f32),
            pltpu.VMEM((CH, DW), f32),
        ] + [pltpu.SemaphoreType.DMA] * 4,
    )
    def _sc_gather(tabs, sd3, out_pq, shared, idx_v, b0, b1,
                   sg0, sg1, ss0, ss1):
        cid = lax.axis_index("c")
        sid = lax.axis_index("s")
        base = sid * (NCH2 * CH)
        pltpu.sync_copy(tabs.at[cid, pl.ds(sid * NPT, NPT)],
                        shared.at[pl.ds(sid * NPT, NPT)])
        pltpu.sync_copy(sd3.at[cid, sid], idx_v)
        plsc.subcore_barrier()

        def wait64(sem):
            pltpu.make_async_copy(out_pq.at[0, pl.ds(0, CH)], b0, sem).wait()

        pltpu.async_copy(shared.at[idx_v.at[0]], b0, sg0)

        def body(i, carry):
            j0 = 2 * i

            @pl.when(i > 0)
            def _():
                wait64(ss1)

            pltpu.async_copy(shared.at[idx_v.at[j0 + 1]], b1, sg1)
            wait64(sg0)
            pltpu.async_copy(b0, out_pq.at[cid, pl.ds(base + j0 * CH, CH)],
                             ss0)

            @pl.when(i < NCH2 // 2 - 1)
            def _():
                wait64(ss0)
                pltpu.async_copy(shared.at[idx_v.at[j0 + 2]], b0, sg0)

            wait64(sg1)
            pltpu.async_copy(b1,
                             out_pq.at[cid, pl.ds(base + (j0 + 1) * CH, CH)],
                             ss1)
            return carry

        lax.fori_loop(0, NCH2 // 2, body, 0)
        wait64(ss0)
        wait64(ss1)

    return _sc_gather


def _gather(tabs, sd3):
    return _sc_gather_call()(tabs, sd3)


@functools.lru_cache(maxsize=None)
def _sc_scatter_call():
    @functools.partial(
        pl.kernel,
        out_type=jax.ShapeDtypeStruct((2, NP, DW), f32),
        mesh=_sc_mesh(),
        scratch_types=[
            pltpu.VMEM_SHARED((NP, DW), f32),
            pltpu.VMEM((NCHS, CH), i32),
            pltpu.VMEM((CH, DW), f32),
            pltpu.VMEM((CH, DW), f32),
            pltpu.SemaphoreType.DMA,
            pltpu.SemaphoreType.DMA,
            pltpu.SemaphoreType.DMA,
            pltpu.SemaphoreType.DMA,
        ],
    )
    def _sc_scatter(rows, src3, acc2, shared, idx_s, b0, b1, sl0, sl1,
                    sa0, sa1):
        cid = lax.axis_index("c")
        sid = lax.axis_index("s")
        wid = cid * 16 + sid
        base = wid * (NCHS * CH)

        def zrow(r, carry):
            for c in range(DW // 16):
                b0[r, pl.ds(c * 16, 16)] = jnp.zeros((16,), f32)
            return carry

        lax.fori_loop(0, CH, zrow, 0)
        for c in range(NPT // CH):
            pltpu.sync_copy(b0, shared.at[pl.ds(sid * NPT + c * CH, CH)])
        plsc.subcore_barrier()

        pltpu.sync_copy(src3.at[wid], idx_s)

        def wait64(sem):
            pltpu.make_async_copy(rows.at[pl.ds(0, CH)], b0, sem).wait()

        pltpu.async_copy(rows.at[pl.ds(base, CH)], b0, sl0)
        pltpu.async_copy(rows.at[pl.ds(base + CH, CH)], b1, sl1)

        def body(i, carry):
            j0 = 2 * i
            wait64(sl0)
            pltpu.async_copy(b0, shared.at[idx_s.at[j0]], sa0, add=True)
            wait64(sl1)
            pltpu.async_copy(b1, shared.at[idx_s.at[j0 + 1]], sa1, add=True)

            @pl.when(i < NCHS // 2 - 1)
            def _():
                wait64(sa0)
                pltpu.async_copy(rows.at[pl.ds(base + (j0 + 2) * CH, CH)],
                                 b0, sl0)
                wait64(sa1)
                pltpu.async_copy(rows.at[pl.ds(base + (j0 + 3) * CH, CH)],
                                 b1, sl1)
            return carry

        lax.fori_loop(0, NCHS // 2, body, 0)
        wait64(sa0)
        wait64(sa1)
        plsc.subcore_barrier()
        pltpu.sync_copy(shared.at[pl.ds(sid * NPT, NPT)],
                        acc2.at[cid, pl.ds(sid * NPT, NPT)])

    return _sc_scatter


def _scatter(rows, src3):
    return _sc_scatter_call()(rows, src3)


def _time_body(sig, tw1t, tb1, tw2t, tb2, w1dt, b1c, out):
    s = sig[0:1, 0:1]
    i = lax.broadcasted_iota(i32, (8, TD // 2), 1).astype(f32)
    freqs = jnp.exp(i * (jnp.log(1000.0) / (TD // 2 - 1)))
    args = s * freqs
    emb = jnp.concatenate([jnp.sin(args), jnp.cos(args)], axis=1)
    te = jax.nn.silu(_dot(emb, tw1t[...]) + tb1[...])
    te = _dot(te, tw2t[...]) + tb2[...]
    out[...] = _dot(te, w1dt[...]) + b1c[...]


def _pro_body(ti, tg, nt, sp, x8,
              pet, net, let, modt, pw1r, pb1, pw2t, pb2, a0t, b0t,
              h_o, t_o):
    tiv = ti[...]
    oh25 = (tiv == lax.broadcasted_iota(i32, (BN, 32), 1)).astype(f32)
    pe = _dot(oh25, pet[...])
    ohn = (jnp.maximum(tiv, 0)
           == lax.broadcasted_iota(i32, (BN, 8), 1)).astype(f32)
    ne = _dot(ohn, net[...])
    oh64 = (tiv == lax.broadcasted_iota(i32, (BN, 64), 1)).astype(f32)
    le = _dot(oh64, let[...])
    tgv = tg[...]
    feats = jnp.where(tgv == 0, pe, 0.0)
    feats = jnp.where(tgv == 1, ne, feats)
    feats = jnp.where(tgv == 2, le, feats)
    ohm = (nt[...] == lax.broadcasted_iota(i32, (BN, 8), 1)).astype(f32)
    feats = feats + _dot(ohm, modt[...])
    p = jax.nn.silu(sp[...] * pw1r[0:1, :] + pb1[0:1, :])
    feats = feats + _dot(p, pw2t[...]) + pb2[0:1, :]
    h_o[...] = feats
    g = _dot(feats, a0t[...])
    k = _dot(feats, b0t[...])
    xs = x8[...][:, 0:3]
    z13 = jnp.zeros((BN, PADW), f32)
    t_o[0] = jnp.concatenate([g, xs, z13], axis=1)
    t_o[1] = jnp.concatenate([k, xs, z13], axis=1)


def _edge_body(pr, qr, et, w2t, cw1t, drbf, ctab, misc, out):
    p = pr[0]
    q = qr[0]
    g = p[:, :H] + q[:, :H]
    rel = p[:, H:H + 3] - q[:, H:H + 3]
    d2 = jnp.sum(rel * rel, axis=1, keepdims=True)
    dist = jnp.maximum(jnp.sqrt(d2), 1e-6)
    centers = (lax.broadcasted_iota(i32, (BE, RBF), 1).astype(f32)
               * (12.0 / (RBF - 1)))
    diff = dist - centers
    rbf = jnp.exp(-0.5 * diff * diff)
    ohe = (et[...] == lax.broadcasted_iota(i32, (BE, 8), 1)).astype(f32)
    pre = g + _dot(rbf, drbf[...]) + _dot(ohe, ctab[...]) + misc[0:1, :]
    m1 = jax.nn.silu(pre)
    msgs = jax.nn.silu(_dot(m1, w2t[...]) + misc[1:2, :])
    c1 = jax.nn.silu(_dot(msgs, cw1t[...]) + misc[2:3, :])
    cs = jnp.sum(c1 * misc[3:4, :], axis=1, keepdims=True) + misc[4:5, 0:1]
    w = cs / (dist + 1.0)
    out[...] = jnp.concatenate(
        [msgs, w * rel, jnp.zeros((BE, PADW), f32)], axis=1)


def _node_common(h, x8, acca, accb, n1ht, n1ft, n2t, nmisc):
    a = (acca[0] + acca[1]) + (accb[0] + accb[1])
    fa = a[:, :H]
    ca = a[:, H:H + 3]
    hv = h[...]
    t1 = jax.nn.silu(_dot(hv, n1ht[...]) + _dot(fa, n1ft[...])
                     + nmisc[0:1, :])
    hn = hv + _dot(t1, n2t[...]) + nmisc[1:2, :]
    xn = x8[...] + jnp.concatenate([ca, jnp.zeros((BN, 5), f32)], axis=1)
    return hn, xn


def _node_body(h, x8, acca, accb, n1ht, n1ft, n2t, nmisc, a1t, b1t,
               h_o, x_o, t_o):
    hn, xn = _node_common(h, x8, acca, accb, n1ht, n1ft, n2t, nmisc)
    h_o[...] = hn
    x_o[...] = xn
    g = _dot(hn, a1t[...])
    k = _dot(hn, b1t[...])
    xs = xn[:, 0:3]
    z13 = jnp.zeros((BN, PADW), f32)
    t_o[0] = jnp.concatenate([g, xs, z13], axis=1)
    t_o[1] = jnp.concatenate([k, xs, z13], axis=1)


def _node_final_body(h, x8, acca, accb, n1ht, n1ft, n2t, nmisc, ow1t,
                     ow2t8, ob8, res_o):
    hn, xn = _node_common(h, x8, acca, accb, n1ht, n1ft, n2t, nmisc)
    d1 = jax.nn.silu(_dot(hn, ow1t[...]) + nmisc[2:3, :])
    res_o[...] = xn + _dot(d1, ow2t8[...]) + ob8[0:1, :]


def _full(shape):
    return pl.BlockSpec(shape, lambda i: (0,) * len(shape))


def _rows(shape):
    return pl.BlockSpec(shape, lambda i: (i,) + (0,) * (len(shape) - 1))


_time_call = pl.pallas_call(
    _time_body,
    out_shape=jax.ShapeDtypeStruct((8, L * H), f32),
)

_pro_call = pl.pallas_call(
    _pro_body,
    grid=(NP // BN,),
    in_specs=[
        _rows((BN, 1)), _rows((BN, 1)), _rows((BN, 1)), _rows((BN, 1)),
        _rows((BN, 8)),
        _full((32, H)), _full((8, H)), _full((64, H)), _full((8, H)),
        _full((8, H)), _full((8, H)), _full((H, H)), _full((8, H)),
        _full((H, H)), _full((H, H)),
    ],
    out_specs=[_rows((BN, H)),
               pl.BlockSpec((2, BN, DW), lambda i: (0, i, 0))],
    out_shape=[jax.ShapeDtypeStruct((NP, H), f32),
               jax.ShapeDtypeStruct((2, NP, DW), f32)],
)

_edge_call = pl.pallas_call(
    _edge_body,
    grid=(EP2 // BE,),
    in_specs=[
        pl.BlockSpec((1, BE, DW), lambda i: (0, i, 0)),
        pl.BlockSpec((1, BE, DW), lambda i: (1, i, 0)),
        _rows((BE, 1)),
        _full((H, H)), _full((H, H)), _full((RBF, H)), _full((8, H)),
        _full((8, H)),
    ],
    out_specs=_rows((BE, DW)),
    out_shape=jax.ShapeDtypeStruct((EP2, DW), f32),
)

_acc_spec = pl.BlockSpec((2, BN, DW), lambda i: (0, i, 0))

_node_call = pl.pallas_call(
    _node_body,
    grid=(NP // BN,),
    in_specs=[
        _rows((BN, H)), _rows((BN, 8)), _acc_spec, _acc_spec,
        _full((H, H)), _full((H, H)), _full((H, H)), _full((8, H)),
        _full((H, H)), _full((H, H)),
    ],
    out_specs=[_rows((BN, H)), _rows((BN, 8)),
               pl.BlockSpec((2, BN, DW), lambda i: (0, i, 0))],
    out_shape=[jax.ShapeDtypeStruct((NP, H), f32),
               jax.ShapeDtypeStruct((NP, 8), f32),
               jax.ShapeDtypeStruct((2, NP, DW), f32)],
)

_node_final_call = pl.pallas_call(
    _node_final_body,
    grid=(NP // BN,),
    in_specs=[
        _rows((BN, H)), _rows((BN, 8)), _acc_spec, _acc_spec,
        _full((H, H)), _full((H, H)), _full((H, H)), _full((8, H)),
        _full((H, H)), _full((H, 8)), _full((8, 8)),
    ],
    out_specs=_rows((BN, 8)),
    out_shape=jax.ShapeDtypeStruct((NP, 8), f32),
)


def kernel(noisy_coords, sigma, token_groups, token_indices, node_types,
           sequence_positions, edge_index, edge_types, params):
    layers = params['layers']

    def bc8(v):
        return jnp.broadcast_to(v[None, :], (8, v.shape[0]))

    x8 = jnp.pad(noisy_coords[0].astype(f32), ((0, NP - N), (0, 5)))
    ti2 = jnp.pad(token_indices.astype(i32), (0, NP - N))[:, None]
    tg2 = jnp.pad(token_groups.astype(i32), (0, NP - N))[:, None]
    nt2 = jnp.pad(node_types.astype(i32), (0, NP - N))[:, None]
    sp2 = jnp.pad(sequence_positions.astype(f32), (0, NP - N))[:, None]
    src = edge_index[:, 0].astype(i32)
    dst = edge_index[:, 1].astype(i32)
    idx_pad = jnp.full((EP - E,), PAD_ROW, i32)
    src_p = jnp.concatenate([src, idx_pad])
    dst_p = jnp.concatenate([dst, idx_pad])
    srcA3 = src_p[:EP2].reshape(NTILES, NCHS, CH)
    srcB3 = src_p[EP2:].reshape(NTILES, NCHS, CH)
    sdA = jnp.stack([src_p[:EP2].reshape(16, NCH2, CH),
                     dst_p[:EP2].reshape(16, NCH2, CH)])
    sdB = jnp.stack([src_p[EP2:].reshape(16, NCH2, CH),
                     dst_p[EP2:].reshape(16, NCH2, CH)])
    et_p = jnp.pad(edge_types.astype(i32), (0, EP - E))[:, None]
    etA = et_p[:EP2]
    etB = et_p[EP2:]

    w1dt = jnp.concatenate(
        [lp['edge_W1'][:, 288:320].T for lp in layers], axis=1)
    b1c = jnp.broadcast_to(
        jnp.concatenate([lp['edge_b1'] for lp in layers])[None, :],
        (8, L * H))
    sig8 = jnp.broadcast_to(sigma.reshape(1, 1).astype(f32), (8, 128))
    tv = _time_call(sig8, params['time_W1'].T, bc8(params['time_b1']),
                    params['time_W2'].T, bc8(params['time_b2']), w1dt, b1c)

    at = [lp['edge_W1'][:, 0:H].T for lp in layers]
    bt = [lp['edge_W1'][:, H:2 * H].T for lp in layers]
    drbf = [lp['edge_W1'][:, 3 * H + TD:].T for lp in layers]
    ctab = [jnp.pad(_dot(lp['edge_emb'], lp['edge_W1'][:, 2 * H:3 * H].T),
                    ((0, 3), (0, 0))) for lp in layers]
    w2t = [lp['edge_W2'].T for lp in layers]
    cw1t = [lp['coord_W1'].T for lp in layers]
    miscs = []
    for l, lp in enumerate(layers):
        miscs.append(jnp.stack([
            tv[0, l * H:(l + 1) * H],
            lp['edge_b2'],
            lp['coord_b1'],
            lp['coord_W2'][0],
            jnp.full((H,), lp['coord_b2'][0]),
            jnp.zeros((H,), f32), jnp.zeros((H,), f32), jnp.zeros((H,), f32),
        ]))
    n1ht = [lp['node_W1'][:, 0:H].T for lp in layers]
    n1ft = [lp['node_W1'][:, H:].T for lp in layers]
    n2t = [lp['node_W2'].T for lp in layers]
    nmiscs = []
    for lp in layers:
        nmiscs.append(jnp.stack([
            lp['node_b1'], lp['node_b2'], params['out_b1'],
            jnp.zeros((H,), f32), jnp.zeros((H,), f32),
            jnp.zeros((H,), f32), jnp.zeros((H,), f32), jnp.zeros((H,), f32),
        ]))
    ow1t = params['out_W1'].T
    ow2t8 = jnp.pad(params['out_W2'].T, ((0, 0), (0, 5)))
    ob8 = jnp.broadcast_to(jnp.pad(params['out_b2'], (0, 5))[None, :], (8, 8))

    h, tabs = _pro_call(
        ti2, tg2, nt2, sp2, x8,
        jnp.pad(params['protein_emb'], ((0, 7), (0, 0))),
        params['nucleotide_emb'],
        params['ligand_emb'],
        jnp.pad(params['modality_emb'], ((0, 5), (0, 0))),
        bc8(params['pos_W1'][:, 0]), bc8(params['pos_b1']),
        params['pos_W2'].T, bc8(params['pos_b2']),
        at[0], bt[0])

    for l in range(L):
        rowsA = _gather(tabs, sdA)
        rowsB = _gather(tabs, sdB)
        mA = _edge_call(rowsA, rowsA, etA, w2t[l], cw1t[l], drbf[l],
                        ctab[l], miscs[l])
        mB = _edge_call(rowsB, rowsB, etB, w2t[l], cw1t[l], drbf[l],
                        ctab[l], miscs[l])
        accA = _scatter(mA, srcA3)
        accB = _scatter(mB, srcB3)
        if l < L - 1:
            h, x8, tabs = _node_call(
                h, x8, accA, accB, n1ht[l], n1ft[l], n2t[l], nmiscs[l],
                at[l + 1], bt[l + 1])
        else:
            res = _node_final_call(
                h, x8, accA, accB, n1ht[l], n1ft[l], n2t[l], nmiscs[l],
                ow1t, ow2t8, ob8)

    return res[:N, :3][None]


# speedup vs baseline: 7.1022x; 1.0951x over previous
import functools

import jax
import jax.numpy as jnp
from jax import lax
from jax.experimental import pallas as pl
from jax.experimental.pallas import tpu as pltpu
from jax.experimental.pallas import tpu_sc as plsc

N, E, H, TD, RBF, L = 10000, 160000, 96, 32, 16, 4
NP = 10240
EP = 163840
DW = 128
PADW = DW - H - 3
PAD_ROW = NP - 2
NTILES, NCHUNK, CH = 32, 40, 128
EPT = NCHUNK * CH
BE = 5120
BN = 1024
NPT = NP // 16

f32 = jnp.float32
i32 = jnp.int32
_dot = functools.partial(jnp.dot, preferred_element_type=jnp.float32)


@functools.lru_cache(maxsize=None)
def _sc_mesh():
    return plsc.VectorSubcoreMesh(
        core_axis_name="c", subcore_axis_name="s",
        num_cores=2, num_subcores=16)


EP2 = EP // 2
NCH2 = 40
NCHS = 20


@functools.lru_cache(maxsize=None)
def _sc_gather_call():
    @functools.partial(
        pl.kernel,
        out_type=jax.ShapeDtypeStruct((2, EP2, DW), f32),
        mesh=_sc_mesh(),
        scratch_types=[
            pltpu.VMEM_SHARED((NP, DW), f32),
            pltpu.VMEM((NCH2, CH), i32),
            pltpu.VMEM((CH, DW), f32),
            pltpu.VMEM((CH, DW), f32),
        ] + [pltpu.SemaphoreType.DMA] * 4,
    )
    def _sc_gather(tabs, sd3, out_pq, shared, idx_v, b0, b1,
                   sg0, sg1, ss0, ss1):
        cid = lax.axis_index("c")
        sid = lax.axis_index("s")
        base = sid * (NCH2 * CH)
        pltpu.sync_copy(tabs.at[cid, pl.ds(sid * NPT, NPT)],
                        shared.at[pl.ds(sid * NPT, NPT)])
        pltpu.sync_copy(sd3.at[cid, sid], idx_v)
        plsc.subcore_barrier()

        def wait64(sem):
            pltpu.make_async_copy(out_pq.at[0, pl.ds(0, CH)], b0, sem).wait()

        pltpu.async_copy(shared.at[idx_v.at[0]], b0, sg0)

        def body(i, carry):
            j0 = 2 * i

            @pl.when(i > 0)
            def _():
                wait64(ss1)

            pltpu.async_copy(shared.at[idx_v.at[j0 + 1]], b1, sg1)
            wait64(sg0)
            pltpu.async_copy(b0, out_pq.at[cid, pl.ds(base + j0 * CH, CH)],
                             ss0)

            @pl.when(i < NCH2 // 2 - 1)
            def _():
                wait64(ss0)
                pltpu.async_copy(shared.at[idx_v.at[j0 + 2]], b0, sg0)

            wait64(sg1)
            pltpu.async_copy(b1,
                             out_pq.at[cid, pl.ds(base + (j0 + 1) * CH, CH)],
                             ss1)
            return carry

        lax.fori_loop(0, NCH2 // 2, body, 0)
        wait64(ss0)
        wait64(ss1)

    return _sc_gather


def _gather(tabs, sd3):
    return _sc_gather_call()(tabs, sd3)


@functools.lru_cache(maxsize=None)
def _sc_scatter_call():
    @functools.partial(
        pl.kernel,
        out_type=jax.ShapeDtypeStruct((2, NP, DW), f32),
        mesh=_sc_mesh(),
        scratch_types=[
            pltpu.VMEM_SHARED((NP, DW), f32),
            pltpu.VMEM((NCHS, CH), i32),
            pltpu.VMEM((CH, DW), f32),
            pltpu.VMEM((CH, DW), f32),
            pltpu.SemaphoreType.DMA,
            pltpu.SemaphoreType.DMA,
            pltpu.SemaphoreType.DMA,
            pltpu.SemaphoreType.DMA,
        ],
    )
    def _sc_scatter(rows, src3, acc2, shared, idx_s, b0, b1, sl0, sl1,
                    sa0, sa1):
        cid = lax.axis_index("c")
        sid = lax.axis_index("s")
        wid = cid * 16 + sid
        base = wid * (NCHS * CH)

        def zrow(r, carry):
            for c in range(DW // 16):
                b0[r, pl.ds(c * 16, 16)] = jnp.zeros((16,), f32)
            return carry

        lax.fori_loop(0, CH, zrow, 0)
        for c in range(NPT // CH):
            pltpu.sync_copy(b0, shared.at[pl.ds(sid * NPT + c * CH, CH)])
        plsc.subcore_barrier()

        pltpu.sync_copy(src3.at[wid], idx_s)

        def wait64(sem):
            pltpu.make_async_copy(rows.at[pl.ds(0, CH)], b0, sem).wait()

        pltpu.async_copy(rows.at[pl.ds(base, CH)], b0, sl0)
        pltpu.async_copy(rows.at[pl.ds(base + CH, CH)], b1, sl1)

        def body(i, carry):
            j0 = 2 * i
            wait64(sl0)
            pltpu.async_copy(b0, shared.at[idx_s.at[j0]], sa0, add=True)
            wait64(sl1)
            pltpu.async_copy(b1, shared.at[idx_s.at[j0 + 1]], sa1, add=True)

            @pl.when(i < NCHS // 2 - 1)
            def _():
                wait64(sa0)
                pltpu.async_copy(rows.at[pl.ds(base + (j0 + 2) * CH, CH)],
                                 b0, sl0)
                wait64(sa1)
                pltpu.async_copy(rows.at[pl.ds(base + (j0 + 3) * CH, CH)],
                                 b1, sl1)
            return carry

        lax.fori_loop(0, NCHS // 2, body, 0)
        wait64(sa0)
        wait64(sa1)
        plsc.subcore_barrier()
        pltpu.sync_copy(shared.at[pl.ds(sid * NPT, NPT)],
                        acc2.at[cid, pl.ds(sid * NPT, NPT)])

    return _sc_scatter


def _scatter(rows, src3):
    return _sc_scatter_call()(rows, src3)


def _time_body(sig, tw1t, tb1, tw2t, tb2, w1dt, b1c, out):
    s = sig[0:1, 0:1]
    i = lax.broadcasted_iota(i32, (8, TD // 2), 1).astype(f32)
    freqs = jnp.exp(i * (jnp.log(1000.0) / (TD // 2 - 1)))
    args = s * freqs
    emb = jnp.concatenate([jnp.sin(args), jnp.cos(args)], axis=1)
    te = jax.nn.silu(_dot(emb, tw1t[...]) + tb1[...])
    te = _dot(te, tw2t[...]) + tb2[...]
    out[...] = _dot(te, w1dt[...]) + b1c[...]


def _pro_body(ti, tg, nt, sp, x8,
              pet, net, let, modt, pw1r, pb1, pw2t, pb2, a0t, b0t,
              h_o, t_o):
    tiv = ti[...]
    oh25 = (tiv == lax.broadcasted_iota(i32, (BN, 32), 1)).astype(f32)
    pe = _dot(oh25, pet[...])
    ohn = (jnp.maximum(tiv, 0)
           == lax.broadcasted_iota(i32, (BN, 8), 1)).astype(f32)
    ne = _dot(ohn, net[...])
    oh64 = (tiv == lax.broadcasted_iota(i32, (BN, 64), 1)).astype(f32)
    le = _dot(oh64, let[...])
    tgv = tg[...]
    feats = jnp.where(tgv == 0, pe, 0.0)
    feats = jnp.where(tgv == 1, ne, feats)
    feats = jnp.where(tgv == 2, le, feats)
    ohm = (nt[...] == lax.broadcasted_iota(i32, (BN, 8), 1)).astype(f32)
    feats = feats + _dot(ohm, modt[...])
    p = jax.nn.silu(sp[...] * pw1r[0:1, :] + pb1[0:1, :])
    feats = feats + _dot(p, pw2t[...]) + pb2[0:1, :]
    h_o[...] = feats
    g = _dot(feats, a0t[...])
    k = _dot(feats, b0t[...])
    xs = x8[...][:, 0:3]
    z13 = jnp.zeros((BN, PADW), f32)
    t_o[0] = jnp.concatenate([g, xs, z13], axis=1)
    t_o[1] = jnp.concatenate([k, xs, z13], axis=1)


def _edge_body(pr, qr, et, w2t, cw1t, drbf, ctab, misc, out):
    p = pr[0]
    q = qr[0]
    g = p[:, :H] + q[:, :H]
    rel = p[:, H:H + 3] - q[:, H:H + 3]
    d2 = jnp.sum(rel * rel, axis=1, keepdims=True)
    dist = jnp.maximum(jnp.sqrt(d2), 1e-6)
    centers = (lax.broadcasted_iota(i32, (BE, RBF), 1).astype(f32)
               * (12.0 / (RBF - 1)))
    diff = dist - centers
    rbf = jnp.exp(-0.5 * diff * diff)
    ohe = (et[...] == lax.broadcasted_iota(i32, (BE, 8), 1)).astype(f32)
    pre = g + _dot(rbf, drbf[...]) + _dot(ohe, ctab[...]) + misc[0:1, :]
    m1 = jax.nn.silu(pre)
    msgs = jax.nn.silu(_dot(m1, w2t[...]) + misc[1:2, :])
    c1 = jax.nn.silu(_dot(msgs, cw1t[...]) + misc[2:3, :])
    cs = jnp.sum(c1 * misc[3:4, :], axis=1, keepdims=True) + misc[4:5, 0:1]
    w = cs / (dist + 1.0)
    out[...] = jnp.concatenate(
        [msgs, w * rel, jnp.zeros((BE, PADW), f32)], axis=1)


def _node_common(h, x8, acca, accb, n1ht, n1ft, n2t, nmisc):
    a = (acca[0] + acca[1]) + (accb[0] + accb[1])
    fa = a[:, :H]
    ca = a[:, H:H + 3]
    hv = h[...]
    t1 = jax.nn.silu(_dot(hv, n1ht[...]) + _dot(fa, n1ft[...])
                     + nmisc[0:1, :])
    hn = hv + _dot(t1, n2t[...]) + nmisc[1:2, :]
    xn = x8[...] + jnp.concatenate([ca, jnp.zeros((BN, 5), f32)], axis=1)
    return hn, xn


def _node_body(h, x8, acca, accb, n1ht, n1ft, n2t, nmisc, a1t, b1t,
               h_o, x_o, t_o):
    hn, xn = _node_common(h, x8, acca, accb, n1ht, n1ft, n2t, nmisc)
    h_o[...] = hn
    x_o[...] = xn
    g = _dot(hn, a1t[...])
    k = _dot(hn, b1t[...])
    xs = xn[:, 0:3]
    z13 = jnp.zeros((BN, PADW), f32)
    t_o[0] = jnp.concatenate([g, xs, z13], axis=1)
    t_o[1] = jnp.concatenate([k, xs, z13], axis=1)


def _node_final_body(h, x8, acca, accb, n1ht, n1ft, n2t, nmisc, ow1t,
                     ow2t8, ob8, res_o):
    hn, xn = _node_common(h, x8, acca, accb, n1ht, n1ft, n2t, nmisc)
    d1 = jax.nn.silu(_dot(hn, ow1t[...]) + nmisc[2:3, :])
    res_o[...] = xn + _dot(d1, ow2t8[...]) + ob8[0:1, :]


def _full(shape):
    return pl.BlockSpec(shape, lambda i: (0,) * len(shape))


def _rows(shape):
    return pl.BlockSpec(shape, lambda i: (i,) + (0,) * (len(shape) - 1))


_time_call = pl.pallas_call(
    _time_body,
    out_shape=jax.ShapeDtypeStruct((8, L * H), f32),
)

_pro_call = pl.pallas_call(
    _pro_body,
    grid=(NP // BN,),
    in_specs=[
        _rows((BN, 1)), _rows((BN, 1)), _rows((BN, 1)), _rows((BN, 1)),
        _rows((BN, 8)),
        _full((32, H)), _full((8, H)), _full((64, H)), _full((8, H)),
        _full((8, H)), _full((8, H)), _full((H, H)), _full((8, H)),
        _full((H, H)), _full((H, H)),
    ],
    out_specs=[_rows((BN, H)),
               pl.BlockSpec((2, BN, DW), lambda i: (0, i, 0))],
    out_shape=[jax.ShapeDtypeStruct((NP, H), f32),
               jax.ShapeDtypeStruct((2, NP, DW), f32)],
)

_edge_call = pl.pallas_call(
    _edge_body,
    grid=(EP2 // BE,),
    in_specs=[
        pl.BlockSpec((1, BE, DW), lambda i: (0, i, 0)),
        pl.BlockSpec((1, BE, DW), lambda i: (1, i, 0)),
        _rows((BE, 1)),
        _full((H, H)), _full((H, H)), _full((RBF, H)), _full((8, H)),
        _full((8, H)),
    ],
    out_specs=_rows((BE, DW)),
    out_shape=jax.ShapeDtypeStruct((EP2, DW), f32),
)

_acc_spec = pl.BlockSpec((2, BN, DW), lambda i: (0, i, 0))

_node_call = pl.pallas_call(
    _node_body,
    grid=(NP // BN,),
    in_specs=[
        _rows((BN, H)), _rows((BN, 8)), _acc_spec, _acc_spec,
        _full((H, H)), _full((H, H)), _full((H, H)), _full((8, H)),
        _full((H, H)), _full((H, H)),
    ],
    out_specs=[_rows((BN, H)), _rows((BN, 8)),
               pl.BlockSpec((2, BN, DW), lambda i: (0, i, 0))],
    out_shape=[jax.ShapeDtypeStruct((NP, H), f32),
               jax.ShapeDtypeStruct((NP, 8), f32),
               jax.ShapeDtypeStruct((2, NP, DW), f32)],
)

_node_final_call = pl.pallas_call(
    _node_final_body,
    grid=(NP // BN,),
    in_specs=[
        _rows((BN, H)), _rows((BN, 8)), _acc_spec, _acc_spec,
        _full((H, H)), _full((H, H)), _full((H, H)), _full((8, H)),
        _full((H, H)), _full((H, 8)), _full((8, 8)),
    ],
    out_specs=_rows((BN, 8)),
    out_shape=jax.ShapeDtypeStruct((NP, 8), f32),
)


def kernel(noisy_coords, sigma, token_groups, token_indices, node_types,
           sequence_positions, edge_index, edge_types, params):
    layers = params['layers']

    def bc8(v):
        return jnp.broadcast_to(v[None, :], (8, v.shape[0]))

    x8 = jnp.pad(noisy_coords[0].astype(f32), ((0, NP - N), (0, 5)))
    ti2 = jnp.pad(token_indices.astype(i32), (0, NP - N))[:, None]
    tg2 = jnp.pad(token_groups.astype(i32), (0, NP - N))[:, None]
    nt2 = jnp.pad(node_types.astype(i32), (0, NP - N))[:, None]
    sp2 = jnp.pad(sequence_positions.astype(f32), (0, NP - N))[:, None]
    src = edge_index[:, 0].astype(i32)
    dst = edge_index[:, 1].astype(i32)
    idx_pad = jnp.full((EP - E,), PAD_ROW, i32)
    src_p = jnp.concatenate([src, idx_pad])
    dst_p = jnp.concatenate([dst, idx_pad])
    srcA3 = src_p[:EP2].reshape(NTILES, NCHS, CH)
    srcB3 = src_p[EP2:].reshape(NTILES, NCHS, CH)
    sdA = jnp.stack([src_p[:EP2].reshape(16, NCH2, CH),
                     dst_p[:EP2].reshape(16, NCH2, CH)])
    sdB = jnp.stack([src_p[EP2:].reshape(16, NCH2, CH),
                     dst_p[EP2:].reshape(16, NCH2, CH)])
    et_p = jnp.pad(edge_types.astype(i32), (0, EP - E))[:, None]
    etA = et_p[:EP2]
    etB = et_p[EP2:]

    w1dt = jnp.concatenate(
        [lp['edge_W1'][:, 288:320].T for lp in layers], axis=1)
    b1c = jnp.broadcast_to(
        jnp.concatenate([lp['edge_b1'] for lp in layers])[None, :],
        (8, L * H))
    sig8 = jnp.broadcast_to(sigma.reshape(1, 1).astype(f32), (8, 128))
    tv = _time_call(sig8, params['time_W1'].T, bc8(params['time_b1']),
                    params['time_W2'].T, bc8(params['time_b2']), w1dt, b1c)

    at = [lp['edge_W1'][:, 0:H].T for lp in layers]
    bt = [lp['edge_W1'][:, H:2 * H].T for lp in layers]
    drbf = [lp['edge_W1'][:, 3 * H + TD:].T for lp in layers]
    ctab = [jnp.pad(_dot(lp['edge_emb'], lp['edge_W1'][:, 2 * H:3 * H].T),
                    ((0, 3), (0, 0))) for lp in layers]
    w2t = [lp['edge_W2'].T for lp in layers]
    cw1t = [lp['coord_W1'].T for lp in layers]
    miscs = []
    for l, lp in enumerate(layers):
        miscs.append(jnp.stack([
            tv[0, l * H:(l + 1) * H],
            lp['edge_b2'],
            lp['coord_b1'],
            lp['coord_W2'][0],
            jnp.full((H,), lp['coord_b2'][0]),
            jnp.zeros((H,), f32), jnp.zeros((H,), f32), jnp.zeros((H,), f32),
        ]))
    n1ht = [lp['node_W1'][:, 0:H].T for lp in layers]
    n1ft = [lp['node_W1'][:, H:].T for lp in layers]
    n2t = [lp['node_W2'].T for lp in layers]
    nmiscs = []
    for lp in layers:
        nmiscs.append(jnp.stack([
            lp['node_b1'], lp['node_b2'], params['out_b1'],
            jnp.zeros((H,), f32), jnp.zeros((H,), f32),
            jnp.zeros((H,), f32), jnp.zeros((H,), f32), jnp.zeros((H,), f32),
        ]))
    ow1t = params['out_W1'].T
    ow2t8 = jnp.pad(params['out_W2'].T, ((0, 0), (0, 5)))
    ob8 = jnp.broadcast_to(jnp.pad(params['out_b2'], (0, 5))[None, :], (8, 8))

    h, tabs = _pro_call(
        ti2, tg2, nt2, sp2, x8,
        jnp.pad(params['protein_emb'], ((0, 7), (0, 0))),
        params['nucleotide_emb'],
        params['ligand_emb'],
        jnp.pad(params['modality_emb'], ((0, 5), (0, 0))),
        bc8(params['pos_W1'][:, 0]), bc8(params['pos_b1']),
        params['pos_W2'].T, bc8(params['pos_b2']),
        at[0], bt[0])

    for l in range(L):
        rowsA = _gather(tabs, sdA)
        rowsB = _gather(tabs, sdB)
        mA = _edge_call(rowsA, rowsA, etA, w2t[l], cw1t[l], drbf[l],
                        ctab[l], miscs[l])
        mB = _edge_call(rowsB, rowsB, etB, w2t[l], cw1t[l], drbf[l],
                        ctab[l], miscs[l])
        accA = _scatter(mA, srcA3)
        accB = _scatter(mB, srcB3)
        if l < L - 1:
            h, x8, tabs = _node_call(
                h, x8, accA, accB, n1ht[l], n1ft[l], n2t[l], nmiscs[l],
                at[l + 1], bt[l + 1])
        else:
            res = _node_final_call(
                h, x8, accA, accB, n1ht[l], n1ft[l], n2t[l], nmiscs[l],
                ow1t, ow2t8, ob8)

    return res[:N, :3][None]
